# Initial kernel scaffold; baseline (speedup 1.0000x reference)
import jax
import jax.numpy as jnp
from jax.experimental import pallas as pl


def kernel(x, edge_index, edge_attr, W1, b1, gamma, beta, W2, b2, Wm, bm, Wx1, bx1, Wx2, We1, be1, We2, be2, We3, be3):
    raise NotImplementedError("write your pallas kernel here")


# speedup vs baseline: 13.2280x; 13.2280x over previous
import functools

import jax
import jax.numpy as jnp
from jax import lax
from jax.experimental import pallas as pl
from jax.experimental.pallas import tpu as pltpu
from jax.experimental.pallas import tpu_sc as plsc

N_NODES = 100000
N_EDGES = 1600000
NH = 32
N_LAYERS = 2
C_WEIGHT = 0.001

NC = 2
NS = 16
NW = NC * NS

G_RANGES = 5
EQ = N_EDGES // G_RANGES
GK = 6400
G_NCHUNK = EQ // GK

SROWS = 56
S_NCHUNK = 7
SR_TILE = S_NCHUNK * SROWS
SR_TOT = NW * SR_TILE
E_PAD = SR_TOT * 128
NP_PAD = 100352
NZ = NP_PAD // NS

TB = 16000
T_GRID = N_EDGES // TB


def _psi(v):
    return jnp.sign(v) * jnp.log(jnp.abs(v) + 1.0)


def _sc_gather(x3, ei3):
    mesh = plsc.VectorSubcoreMesh(core_axis_name="c", subcore_axis_name="s")

    @functools.partial(
        pl.kernel,
        out_type=jax.ShapeDtypeStruct((6, 1, N_EDGES), jnp.float32),
        mesh=mesh,
        compiler_params=pltpu.CompilerParams(use_tc_tiling_on_sc=False, needs_layout_passes=False),
        scratch_types=[
            pltpu.VMEM((N_NODES,), jnp.float32),
            pltpu.VMEM((GK,), jnp.int32),
            pltpu.VMEM((GK,), jnp.float32),
        ],
    )
    def gather_k(x_hbm, ei_hbm, out_hbm, col_v, idx_v, val_v):
        wid = lax.axis_index("s") * NC + lax.axis_index("c")

        @pl.when(wid < 2 * 3 * G_RANGES)
        def _():
            side = wid // (3 * G_RANGES)
            sub = wid % (3 * G_RANGES)
            coord = sub % 3
            rng = sub // 3
            pltpu.sync_copy(x_hbm.at[coord, 0], col_v)

            @pl.loop(0, G_NCHUNK)
            def _chunk(ci):
                base = rng * EQ + ci * GK
                pltpu.sync_copy(ei_hbm.at[side, 0, pl.ds(base, GK)], idx_v)

                @pl.loop(0, GK // 16, unroll=4)
                def _vec(j):
                    iv = idx_v[pl.ds(j * 16, 16)]
                    val_v[pl.ds(j * 16, 16)] = plsc.load_gather(col_v, [iv])

                pltpu.sync_copy(
                    val_v, out_hbm.at[side * 3 + coord, 0, pl.ds(base, GK)])

    return gather_k(x3, ei3)


def _sc_scatter(upd3, src2d, zeros_row):
    mesh = plsc.VectorSubcoreMesh(core_axis_name="c", subcore_axis_name="s")

    @functools.partial(
        pl.kernel,
        out_type=jax.ShapeDtypeStruct((6, 1, NP_PAD), jnp.float32),
        mesh=mesh,
        compiler_params=pltpu.CompilerParams(use_tc_tiling_on_sc=False, needs_layout_passes=False),
        scratch_types=[
            pltpu.VMEM_SHARED((NP_PAD,), jnp.float32),
            pltpu.VMEM_SHARED((NP_PAD,), jnp.float32),
            pltpu.VMEM_SHARED((NP_PAD,), jnp.float32),
            pltpu.VMEM((SROWS, 128), jnp.int32),
            pltpu.VMEM((SROWS, 128), jnp.float32),
        ],
    )
    def scatter_k(upd_hbm, src_hbm, zrow_hbm, out_hbm, sh0, sh1, sh2,
                  idx_v, val_v):
        cid = lax.axis_index("c")
        sid = lax.axis_index("s")
        wid = sid * NC + cid
        shared = (sh0, sh1, sh2)

        for c in range(3):
            pltpu.sync_copy(zrow_hbm.at[pl.ds(sid * NZ, NZ)],
                            shared[c].at[pl.ds(sid * NZ, NZ)])
        plsc.subcore_barrier()

        @pl.loop(0, S_NCHUNK)
        def _chunk(t):
            rbase = wid * SR_TILE + t * SROWS
            pltpu.sync_copy(src_hbm.at[pl.ds(rbase, SROWS)], idx_v)
            for c in range(3):
                pltpu.sync_copy(upd_hbm.at[c, pl.ds(rbase, SROWS)], val_v)

                @pl.loop(0, SROWS)
                def _row(r):
                    pltpu.sync_copy(val_v.at[r],
                                    shared[c].at[idx_v.at[r]], add=True)

        plsc.subcore_barrier()
        for c in range(3):
            pltpu.sync_copy(shared[c].at[pl.ds(sid * NZ, NZ)],
                            out_hbm.at[cid * 3 + c, 0, pl.ds(sid * NZ, NZ)])

    return scatter_k(upd3, src2d, zeros_row)


def _edge_features(xsd_ref, ea_ref):
    xs = xsd_ref[0:3, 0]
    xd = xsd_ref[3:6, 0]
    dif = xs - xd
    norms = _psi(jnp.sum(dif * dif, axis=0, keepdims=True))
    dots = _psi(jnp.sum(xs * xd, axis=0, keepdims=True))
    m_in = jnp.concatenate([xd, xs, ea_ref[...], norms, dots], axis=0)
    return m_in, dif


def _tc_stats(xsd, ea_t, w1t):
    def body(xsd_ref, ea_ref, w1t_ref, out_ref):
        i = pl.program_id(0)

        @pl.when(i == 0)
        def _():
            out_ref[...] = jnp.zeros_like(out_ref)

        m_in, _ = _edge_features(xsd_ref, ea_ref)
        h0 = jnp.dot(w1t_ref[...], m_in, preferred_element_type=jnp.float32)
        out_ref[:, 0:1] += jnp.sum(h0, axis=1, keepdims=True)
        out_ref[:, 1:2] += jnp.sum(h0 * h0, axis=1, keepdims=True)

    return pl.pallas_call(
        body,
        grid=(T_GRID,),
        in_specs=[
            pl.BlockSpec((6, 1, TB), lambda i: (0, 0, i)),
            pl.BlockSpec((4, TB), lambda i: (0, i)),
            pl.BlockSpec((NH, 12), lambda i: (0, 0)),
        ],
        out_specs=pl.BlockSpec((NH, 128), lambda i: (0, 0)),
        out_shape=jax.ShapeDtypeStruct((NH, 128), jnp.float32),
    )(xsd, ea_t, w1t)


def _tc_mlp(xsd, ea_t, w1t, scale, shift, w2ct, b2c, wx1t, bx1, wx2t):
    def body(xsd_ref, ea_ref, w1t_ref, scale_ref, shift_ref, w2ct_ref,
             b2c_ref, wx1t_ref, bx1_ref, wx2t_ref, out_ref):
        m_in, dif = _edge_features(xsd_ref, ea_ref)
        h0 = jnp.dot(w1t_ref[...], m_in, preferred_element_type=jnp.float32)
        h1 = jax.nn.relu(h0 * scale_ref[...] + shift_ref[...])
        z = jnp.dot(w2ct_ref[...], h1,
                    preferred_element_type=jnp.float32) + b2c_ref[...]
        h2 = jax.nn.relu(z[0:NH])
        wgt = jax.nn.sigmoid(z[NH:NH + 1])
        m = h2 * wgt
        p = jax.nn.relu(
            jnp.dot(wx1t_ref[...], m,
                    preferred_element_type=jnp.float32) + bx1_ref[...])
        px = jnp.dot(wx2t_ref[...], p, preferred_element_type=jnp.float32)
        out_ref[...] = jnp.clip(dif * px, -100.0, 100.0)

    return pl.pallas_call(
        body,
        grid=(T_GRID,),
        in_specs=[
            pl.BlockSpec((6, 1, TB), lambda i: (0, 0, i)),
            pl.BlockSpec((4, TB), lambda i: (0, i)),
            pl.BlockSpec((NH, 12), lambda i: (0, 0)),
            pl.BlockSpec((NH, 1), lambda i: (0, 0)),
            pl.BlockSpec((NH, 1), lambda i: (0, 0)),
            pl.BlockSpec((NH + 1, NH), lambda i: (0, 0)),
            pl.BlockSpec((NH + 1, 1), lambda i: (0, 0)),
            pl.BlockSpec((NH, NH), lambda i: (0, 0)),
            pl.BlockSpec((NH, 1), lambda i: (0, 0)),
            pl.BlockSpec((1, NH), lambda i: (0, 0)),
        ],
        out_specs=pl.BlockSpec((3, TB), lambda i: (0, i)),
        out_shape=jax.ShapeDtypeStruct((3, N_EDGES), jnp.float32),
    )(xsd, ea_t, w1t, scale, shift, w2ct, b2c, wx1t, bx1, wx2t)


def _tc_final(xsd, we1t, be1, we2t, be2, we3t, be3):
    def body(xsd_ref, we1t_ref, be1_ref, we2t_ref, be2_ref, we3t_ref,
             be3_ref, out_ref):
        cat = jnp.concatenate([xsd_ref[3:6, 0], xsd_ref[0:3, 0]], axis=0)
        o1 = jax.nn.relu(
            jnp.dot(we1t_ref[...], cat,
                    preferred_element_type=jnp.float32) + be1_ref[...])
        o2 = jax.nn.relu(
            jnp.dot(we2t_ref[...], o1,
                    preferred_element_type=jnp.float32) + be2_ref[...])
        o3 = jnp.dot(we3t_ref[...], o2,
                     preferred_element_type=jnp.float32) + be3_ref[...]
        out_ref[...] = jax.nn.sigmoid(o3)

    return pl.pallas_call(
        body,
        grid=(T_GRID,),
        in_specs=[
            pl.BlockSpec((6, 1, TB), lambda i: (0, 0, i)),
            pl.BlockSpec((NH, 6), lambda i: (0, 0)),
            pl.BlockSpec((NH, 1), lambda i: (0, 0)),
            pl.BlockSpec((NH, NH), lambda i: (0, 0)),
            pl.BlockSpec((NH, 1), lambda i: (0, 0)),
            pl.BlockSpec((1, NH), lambda i: (0, 0)),
            pl.BlockSpec((1, 1), lambda i: (0, 0)),
        ],
        out_specs=pl.BlockSpec((1, TB), lambda i: (0, i)),
        out_shape=jax.ShapeDtypeStruct((1, N_EDGES), jnp.float32),
    )(xsd, we1t, be1, we2t, be2, we3t, be3)


def kernel(x, edge_index, edge_attr, W1, b1, gamma, beta, W2, b2, Wm, bm,
           Wx1, bx1, Wx2, We1, be1, We2, be2, We3, be3):
    f32 = jnp.float32
    x3 = x.T[:, None, :]
    ei3 = edge_index[:, None, :]
    ea_t = edge_attr.T
    zeros_row = jnp.zeros((NP_PAD,), f32)
    src2d = jnp.concatenate(
        [edge_index[0],
         jnp.full((E_PAD - N_EDGES,), N_NODES, jnp.int32)]).reshape(
             SR_TOT, 128)

    for l in range(N_LAYERS):
        xsd = _sc_gather(x3, ei3)
        w1t = W1[l].T
        mom = _tc_stats(xsd, ea_t, w1t)
        s1 = mom[:, 0:1] / N_EDGES
        s2 = mom[:, 1:2] / N_EDGES
        mu = s1 + b1[l][:, None]
        var = s2 - s1 * s1
        scale = gamma[l][:, None] * lax.rsqrt(var + 1e-5)
        shift = beta[l][:, None] - (mu * scale)
        w2ct = jnp.concatenate([W2[l], Wm[l]], axis=1).T
        b2c = jnp.concatenate([b2[l], bm[l]])[:, None]
        upd_t = _tc_mlp(xsd, ea_t, w1t, scale, shift, w2ct, b2c,
                        Wx1[l].T, bx1[l][:, None], Wx2[l].T)
        upd3 = jnp.concatenate(
            [upd_t, jnp.zeros((3, E_PAD - N_EDGES), f32)],
            axis=1).reshape(3, SR_TOT, 128)
        partials = _sc_scatter(upd3, src2d, zeros_row)
        pr = partials.reshape(2, 3, NP_PAD)
        agg = (pr[0] + pr[1])[:, :N_NODES]
        x3 = x3 + C_WEIGHT * agg[:, None, :]

    xsd = _sc_gather(x3, ei3)
    out = _tc_final(xsd, We1.T, be1[:, None], We2.T, be2[:, None],
                    We3.T, be3[None, :])
    return out.reshape(N_EDGES, 1)


# speedup vs baseline: 15.5715x; 1.1772x over previous
import functools

import jax
import jax.numpy as jnp
from jax import lax
from jax.experimental import pallas as pl
from jax.experimental.pallas import tpu as pltpu
from jax.experimental.pallas import tpu_sc as plsc

N_NODES = 100000
N_EDGES = 1600000
NH = 32
N_LAYERS = 2
C_WEIGHT = 0.001

NC = 2
NS = 16
NW = NC * NS

G_RANGES = 5
EQ = N_EDGES // G_RANGES
GK = 6400
G_NCHUNK = EQ // GK

SK = 6272
S_NCHUNK = 8
S_TILE = S_NCHUNK * SK
E_PAD = NW * S_TILE
NP_PAD = 100352
NZ = NP_PAD // NS

TB = 16000
T_GRID = N_EDGES // TB


def _psi(v):
    return jnp.sign(v) * jnp.log(jnp.abs(v) + 1.0)


def _sc_gather(x3, ei3):
    mesh = plsc.VectorSubcoreMesh(core_axis_name="c", subcore_axis_name="s")

    @functools.partial(
        pl.kernel,
        out_type=jax.ShapeDtypeStruct((6, 1, N_EDGES), jnp.float32),
        mesh=mesh,
        compiler_params=pltpu.CompilerParams(use_tc_tiling_on_sc=False, needs_layout_passes=False),
        scratch_types=[
            pltpu.VMEM((N_NODES,), jnp.float32),
            pltpu.VMEM((2, GK), jnp.int32),
            pltpu.VMEM((2, GK), jnp.float32),
            pltpu.SemaphoreType.DMA,
            pltpu.SemaphoreType.DMA,
            pltpu.SemaphoreType.DMA,
            pltpu.SemaphoreType.DMA,
        ],
    )
    def gather_k(x_hbm, ei_hbm, out_hbm, col_v, idx_v, val_v,
                 isem0, isem1, osem0, osem1):
        wid = lax.axis_index("s") * NC + lax.axis_index("c")

        @pl.when(wid < 2 * 3 * G_RANGES)
        def _():
            side = wid // (3 * G_RANGES)
            sub = wid % (3 * G_RANGES)
            coord = sub % 3
            rng = sub // 3
            out6 = side * 3 + coord
            isems = (isem0, isem1)
            osems = (osem0, osem1)

            def idx_dma(ci, b):
                return pltpu.make_async_copy(
                    ei_hbm.at[side, 0, pl.ds(rng * EQ + ci * GK, GK)],
                    idx_v.at[b], isems[b])

            def out_dma(ci, b):
                return pltpu.make_async_copy(
                    val_v.at[b],
                    out_hbm.at[out6, 0, pl.ds(rng * EQ + ci * GK, GK)],
                    osems[b])

            pltpu.sync_copy(x_hbm.at[coord, 0], col_v)
            idx_dma(0, 0).start()
            idx_dma(1, 1).start()

            @pl.loop(0, G_NCHUNK // 2)
            def _pair(t):
                for b in range(2):
                    ci = 2 * t + b
                    idx_dma(ci, b).wait()

                    @pl.when(ci >= 2)
                    def _():
                        out_dma(ci - 2, b).wait()

                    @pl.loop(0, GK // 16, unroll=4)
                    def _vec(j):
                        iv = idx_v[b, pl.ds(j * 16, 16)]
                        val_v[b, pl.ds(j * 16, 16)] = plsc.load_gather(
                            col_v, [iv])

                    out_dma(ci, b).start()

                    @pl.when(ci + 2 < G_NCHUNK)
                    def _():
                        idx_dma(ci + 2, b).start()

            out_dma(G_NCHUNK - 2, 0).wait()
            out_dma(G_NCHUNK - 1, 1).wait()

    return gather_k(x3, ei3)


def _sc_scatter(upd, src1d, zeros_row):
    mesh = plsc.VectorSubcoreMesh(core_axis_name="c", subcore_axis_name="s")

    @functools.partial(
        pl.kernel,
        out_type=jax.ShapeDtypeStruct((6, 1, NP_PAD), jnp.float32),
        mesh=mesh,
        compiler_params=pltpu.CompilerParams(use_tc_tiling_on_sc=False, needs_layout_passes=False),
        scratch_types=[
            pltpu.VMEM_SHARED((NP_PAD,), jnp.float32),
            pltpu.VMEM_SHARED((NP_PAD,), jnp.float32),
            pltpu.VMEM_SHARED((NP_PAD,), jnp.float32),
            pltpu.VMEM((2, SK), jnp.int32),
            pltpu.VMEM((2, 3, SK), jnp.float32),
            pltpu.SemaphoreType.DMA,
            pltpu.SemaphoreType.DMA,
            pltpu.SemaphoreType.DMA,
            pltpu.SemaphoreType.DMA,
            pltpu.SemaphoreType.DMA,
        ],
    )
    def scatter_k(upd_hbm, src_hbm, zrow_hbm, out_hbm, sh0, sh1, sh2,
                  idx_v, val_v, isem0, isem1, vsem0, vsem1, asem):
        cid = lax.axis_index("c")
        sid = lax.axis_index("s")
        wid = sid * NC + cid
        shared = (sh0, sh1, sh2)
        isems = (isem0, isem1)
        vsems = (vsem0, vsem1)

        def idx_dma(ci, b):
            return pltpu.make_async_copy(
                src_hbm.at[pl.ds(wid * S_TILE + ci * SK, SK)],
                idx_v.at[b], isems[b])

        def val_dma(ci, b, c):
            return pltpu.make_async_copy(
                upd_hbm.at[c, pl.ds(wid * S_TILE + ci * SK, SK)],
                val_v.at[b, c], vsems[b])

        for c in range(3):
            pltpu.sync_copy(zrow_hbm.at[pl.ds(sid * NZ, NZ)],
                            shared[c].at[pl.ds(sid * NZ, NZ)])
        plsc.subcore_barrier()

        for b in range(2):
            idx_dma(b, b).start()
            for c in range(3):
                val_dma(b, b, c).start()

        @pl.loop(0, S_NCHUNK // 2)
        def _pair(t):
            for b in range(2):
                ci = 2 * t + b
                idx_dma(ci, b).wait()
                adds = []
                for c in range(3):
                    val_dma(ci, b, c).wait()
                    adds.append(pltpu.make_async_copy(
                        val_v.at[b, c], shared[c].at[idx_v.at[b]], asem))
                for d in adds:
                    d.start(add=True)
                for d in adds:
                    d.wait()

                @pl.when(ci + 2 < S_NCHUNK)
                def _():
                    idx_dma(ci + 2, b).start()
                    for c in range(3):
                        val_dma(ci + 2, b, c).start()

        plsc.subcore_barrier()
        for c in range(3):
            pltpu.sync_copy(shared[c].at[pl.ds(sid * NZ, NZ)],
                            out_hbm.at[cid * 3 + c, 0, pl.ds(sid * NZ, NZ)])

    return scatter_k(upd, src1d, zeros_row)


def _edge_features(xsd_ref, ea_ref):
    xs = xsd_ref[0:3, 0]
    xd = xsd_ref[3:6, 0]
    dif = xs - xd
    norms = _psi(jnp.sum(dif * dif, axis=0, keepdims=True))
    dots = _psi(jnp.sum(xs * xd, axis=0, keepdims=True))
    m_in = jnp.concatenate([xd, xs, ea_ref[...], norms, dots], axis=0)
    return m_in, dif


def _tc_stats(xsd, ea_t, w1t):
    def body(xsd_ref, ea_ref, w1t_ref, out_ref):
        i = pl.program_id(0)

        @pl.when(i == 0)
        def _():
            out_ref[...] = jnp.zeros_like(out_ref)

        m_in, _ = _edge_features(xsd_ref, ea_ref)
        h0 = jnp.dot(w1t_ref[...], m_in, preferred_element_type=jnp.float32)
        out_ref[:, 0:1] += jnp.sum(h0, axis=1, keepdims=True)
        out_ref[:, 1:2] += jnp.sum(h0 * h0, axis=1, keepdims=True)

    return pl.pallas_call(
        body,
        grid=(T_GRID,),
        in_specs=[
            pl.BlockSpec((6, 1, TB), lambda i: (0, 0, i)),
            pl.BlockSpec((4, TB), lambda i: (0, i)),
            pl.BlockSpec((NH, 12), lambda i: (0, 0)),
        ],
        out_specs=pl.BlockSpec((NH, 128), lambda i: (0, 0)),
        out_shape=jax.ShapeDtypeStruct((NH, 128), jnp.float32),
    )(xsd, ea_t, w1t)


def _tc_mlp(xsd, ea_t, w1t, scale, shift, w2ct, b2c, wx1t, bx1, wx2t):
    def body(xsd_ref, ea_ref, w1t_ref, scale_ref, shift_ref, w2ct_ref,
             b2c_ref, wx1t_ref, bx1_ref, wx2t_ref, out_ref):
        m_in, dif = _edge_features(xsd_ref, ea_ref)
        h0 = jnp.dot(w1t_ref[...], m_in, preferred_element_type=jnp.float32)
        h1 = jax.nn.relu(h0 * scale_ref[...] + shift_ref[...])
        z = jnp.dot(w2ct_ref[...], h1,
                    preferred_element_type=jnp.float32) + b2c_ref[...]
        h2 = jax.nn.relu(z[0:NH])
        wgt = jax.nn.sigmoid(z[NH:NH + 1])
        m = h2 * wgt
        p = jax.nn.relu(
            jnp.dot(wx1t_ref[...], m,
                    preferred_element_type=jnp.float32) + bx1_ref[...])
        px = jnp.dot(wx2t_ref[...], p, preferred_element_type=jnp.float32)
        out_ref[...] = jnp.clip(dif * px, -100.0, 100.0)

    return pl.pallas_call(
        body,
        grid=(T_GRID,),
        in_specs=[
            pl.BlockSpec((6, 1, TB), lambda i: (0, 0, i)),
            pl.BlockSpec((4, TB), lambda i: (0, i)),
            pl.BlockSpec((NH, 12), lambda i: (0, 0)),
            pl.BlockSpec((NH, 1), lambda i: (0, 0)),
            pl.BlockSpec((NH, 1), lambda i: (0, 0)),
            pl.BlockSpec((NH + 1, NH), lambda i: (0, 0)),
            pl.BlockSpec((NH + 1, 1), lambda i: (0, 0)),
            pl.BlockSpec((NH, NH), lambda i: (0, 0)),
            pl.BlockSpec((NH, 1), lambda i: (0, 0)),
            pl.BlockSpec((1, NH), lambda i: (0, 0)),
        ],
        out_specs=pl.BlockSpec((3, TB), lambda i: (0, i)),
        out_shape=jax.ShapeDtypeStruct((3, E_PAD), jnp.float32),
    )(xsd, ea_t, w1t, scale, shift, w2ct, b2c, wx1t, bx1, wx2t)


def _tc_final(xsd, we1t, be1, we2t, be2, we3t, be3):
    def body(xsd_ref, we1t_ref, be1_ref, we2t_ref, be2_ref, we3t_ref,
             be3_ref, out_ref):
        cat = jnp.concatenate([xsd_ref[3:6, 0], xsd_ref[0:3, 0]], axis=0)
        o1 = jax.nn.relu(
            jnp.dot(we1t_ref[...], cat,
                    preferred_element_type=jnp.float32) + be1_ref[...])
        o2 = jax.nn.relu(
            jnp.dot(we2t_ref[...], o1,
                    preferred_element_type=jnp.float32) + be2_ref[...])
        o3 = jnp.dot(we3t_ref[...], o2,
                     preferred_element_type=jnp.float32) + be3_ref[...]
        out_ref[...] = jax.nn.sigmoid(o3)

    return pl.pallas_call(
        body,
        grid=(T_GRID,),
        in_specs=[
            pl.BlockSpec((6, 1, TB), lambda i: (0, 0, i)),
            pl.BlockSpec((NH, 6), lambda i: (0, 0)),
            pl.BlockSpec((NH, 1), lambda i: (0, 0)),
            pl.BlockSpec((NH, NH), lambda i: (0, 0)),
            pl.BlockSpec((NH, 1), lambda i: (0, 0)),
            pl.BlockSpec((1, NH), lambda i: (0, 0)),
            pl.BlockSpec((1, 1), lambda i: (0, 0)),
        ],
        out_specs=pl.BlockSpec((1, TB), lambda i: (0, i)),
        out_shape=jax.ShapeDtypeStruct((1, N_EDGES), jnp.float32),
    )(xsd, we1t, be1, we2t, be2, we3t, be3)


def kernel(x, edge_index, edge_attr, W1, b1, gamma, beta, W2, b2, Wm, bm,
           Wx1, bx1, Wx2, We1, be1, We2, be2, We3, be3):
    f32 = jnp.float32
    x3 = x.T[:, None, :]
    ei3 = edge_index[:, None, :]
    ea_t = edge_attr.T
    zeros_row = jnp.zeros((NP_PAD,), f32)
    src1d = jnp.concatenate(
        [edge_index[0], jnp.full((E_PAD - N_EDGES,), N_NODES, jnp.int32)])

    for l in range(N_LAYERS):
        xsd = _sc_gather(x3, ei3)
        w1t = W1[l].T
        mom = _tc_stats(xsd, ea_t, w1t)
        s1 = mom[:, 0:1] / N_EDGES
        s2 = mom[:, 1:2] / N_EDGES
        mu = s1 + b1[l][:, None]
        var = s2 - s1 * s1
        scale = gamma[l][:, None] * lax.rsqrt(var + 1e-5)
        shift = beta[l][:, None] - (mu * scale)
        w2ct = jnp.concatenate([W2[l], Wm[l]], axis=1).T
        b2c = jnp.concatenate([b2[l], bm[l]])[:, None]
        upd = _tc_mlp(xsd, ea_t, w1t, scale, shift, w2ct, b2c,
                      Wx1[l].T, bx1[l][:, None], Wx2[l].T)
        partials = _sc_scatter(upd, src1d, zeros_row)
        pr = partials.reshape(2, 3, NP_PAD)
        agg = (pr[0] + pr[1])[:, :N_NODES]
        x3 = x3 + C_WEIGHT * agg[:, None, :]

    xsd = _sc_gather(x3, ei3)
    out = _tc_final(xsd, We1.T, be1[:, None], We2.T, be2[:, None],
                    We3.T, be3[None, :])
    return out.reshape(N_EDGES, 1)


# speedup vs baseline: 16.3673x; 1.0511x over previous
import functools

import jax
import jax.numpy as jnp
from jax import lax
from jax.experimental import pallas as pl
from jax.experimental.pallas import tpu as pltpu
from jax.experimental.pallas import tpu_sc as plsc

N_NODES = 100000
N_EDGES = 1600000
NH = 32
N_LAYERS = 2
C_WEIGHT = 0.001

NC = 2
NS = 16
NW = NC * NS

G_RANGES = 5
EQ = N_EDGES // G_RANGES
GK = 6400
G_NCHUNK = EQ // GK

SK = 6272
S_NCHUNK = 8
S_TILE = S_NCHUNK * SK
E_PAD = NW * S_TILE
NP_PAD = 100352
NZ = NP_PAD // NS

TB = 16000
T_GRID = N_EDGES // TB


def _psi(v):
    return jnp.sign(v) * jnp.log(jnp.abs(v) + 1.0)


def _sc_gather(x3, ei3):
    mesh = plsc.VectorSubcoreMesh(core_axis_name="c", subcore_axis_name="s")

    @functools.partial(
        pl.kernel,
        out_type=jax.ShapeDtypeStruct((6, 1, N_EDGES), jnp.float32),
        mesh=mesh,
        compiler_params=pltpu.CompilerParams(use_tc_tiling_on_sc=False, needs_layout_passes=False),
        scratch_types=[
            pltpu.VMEM((N_NODES,), jnp.float32),
            pltpu.VMEM((2, GK), jnp.int32),
            pltpu.VMEM((2, GK), jnp.float32),
            pltpu.SemaphoreType.DMA,
            pltpu.SemaphoreType.DMA,
            pltpu.SemaphoreType.DMA,
            pltpu.SemaphoreType.DMA,
        ],
    )
    def gather_k(x_hbm, ei_hbm, out_hbm, col_v, idx_v, val_v,
                 isem0, isem1, osem0, osem1):
        wid = lax.axis_index("s") * NC + lax.axis_index("c")

        @pl.when(wid < 2 * 3 * G_RANGES)
        def _():
            side = wid // (3 * G_RANGES)
            sub = wid % (3 * G_RANGES)
            coord = sub % 3
            rng = sub // 3
            out6 = side * 3 + coord
            isems = (isem0, isem1)
            osems = (osem0, osem1)

            def idx_dma(ci, b):
                return pltpu.make_async_copy(
                    ei_hbm.at[side, 0, pl.ds(rng * EQ + ci * GK, GK)],
                    idx_v.at[b], isems[b])

            def out_dma(ci, b):
                return pltpu.make_async_copy(
                    val_v.at[b],
                    out_hbm.at[out6, 0, pl.ds(rng * EQ + ci * GK, GK)],
                    osems[b])

            pltpu.sync_copy(x_hbm.at[coord, 0], col_v)
            idx_dma(0, 0).start()
            idx_dma(1, 1).start()

            @pl.loop(0, G_NCHUNK // 2)
            def _pair(t):
                for b in range(2):
                    ci = 2 * t + b
                    idx_dma(ci, b).wait()

                    @pl.when(ci >= 2)
                    def _():
                        out_dma(ci - 2, b).wait()

                    @pl.loop(0, GK // 16, unroll=8)
                    def _vec(j):
                        iv = idx_v[b, pl.ds(j * 16, 16)]
                        val_v[b, pl.ds(j * 16, 16)] = plsc.load_gather(
                            col_v, [iv])

                    out_dma(ci, b).start()

                    @pl.when(ci + 2 < G_NCHUNK)
                    def _():
                        idx_dma(ci + 2, b).start()

            out_dma(G_NCHUNK - 2, 0).wait()
            out_dma(G_NCHUNK - 1, 1).wait()

    return gather_k(x3, ei3)


def _sc_scatter(upd, src1d, zeros_row):
    mesh = plsc.VectorSubcoreMesh(core_axis_name="c", subcore_axis_name="s")

    @functools.partial(
        pl.kernel,
        out_type=jax.ShapeDtypeStruct((6, 1, NP_PAD), jnp.float32),
        mesh=mesh,
        compiler_params=pltpu.CompilerParams(use_tc_tiling_on_sc=False, needs_layout_passes=False),
        scratch_types=[
            pltpu.VMEM_SHARED((NP_PAD,), jnp.float32),
            pltpu.VMEM_SHARED((NP_PAD,), jnp.float32),
            pltpu.VMEM_SHARED((NP_PAD,), jnp.float32),
            pltpu.VMEM((2, SK), jnp.int32),
            pltpu.VMEM((2, 3, SK), jnp.float32),
            pltpu.SemaphoreType.DMA,
            pltpu.SemaphoreType.DMA,
            pltpu.SemaphoreType.DMA,
            pltpu.SemaphoreType.DMA,
            pltpu.SemaphoreType.DMA,
        ],
    )
    def scatter_k(upd_hbm, src_hbm, zrow_hbm, out_hbm, sh0, sh1, sh2,
                  idx_v, val_v, isem0, isem1, vsem0, vsem1, asem):
        cid = lax.axis_index("c")
        sid = lax.axis_index("s")
        wid = sid * NC + cid
        shared = (sh0, sh1, sh2)
        isems = (isem0, isem1)
        vsems = (vsem0, vsem1)

        def idx_dma(ci, b):
            return pltpu.make_async_copy(
                src_hbm.at[pl.ds(wid * S_TILE + ci * SK, SK)],
                idx_v.at[b], isems[b])

        def val_dma(ci, b, c):
            return pltpu.make_async_copy(
                upd_hbm.at[c, pl.ds(wid * S_TILE + ci * SK, SK)],
                val_v.at[b, c], vsems[b])

        for c in range(3):
            pltpu.sync_copy(zrow_hbm.at[pl.ds(sid * NZ, NZ)],
                            shared[c].at[pl.ds(sid * NZ, NZ)])
        plsc.subcore_barrier()

        for b in range(2):
            idx_dma(b, b).start()
            for c in range(3):
                val_dma(b, b, c).start()

        @pl.loop(0, S_NCHUNK // 2)
        def _pair(t):
            for b in range(2):
                ci = 2 * t + b
                idx_dma(ci, b).wait()
                adds = []
                for c in range(3):
                    val_dma(ci, b, c).wait()
                    adds.append(pltpu.make_async_copy(
                        val_v.at[b, c], shared[c].at[idx_v.at[b]], asem))
                for d in adds:
                    d.start(add=True)
                for d in adds:
                    d.wait()

                @pl.when(ci + 2 < S_NCHUNK)
                def _():
                    idx_dma(ci + 2, b).start()
                    for c in range(3):
                        val_dma(ci + 2, b, c).start()

        plsc.subcore_barrier()
        for c in range(3):
            pltpu.sync_copy(shared[c].at[pl.ds(sid * NZ, NZ)],
                            out_hbm.at[cid * 3 + c, 0, pl.ds(sid * NZ, NZ)])

    return scatter_k(upd, src1d, zeros_row)


def _edge_features(xsd_ref, ea_ref):
    xs = xsd_ref[0:3, 0]
    xd = xsd_ref[3:6, 0]
    dif = xs - xd
    norms = _psi(jnp.sum(dif * dif, axis=0, keepdims=True))
    dots = _psi(jnp.sum(xs * xd, axis=0, keepdims=True))
    m_in = jnp.concatenate([xd, xs, ea_ref[...], norms, dots], axis=0)
    return m_in, dif


def _tc_stats(xsd, ea_t, w1t):
    def body(xsd_ref, ea_ref, w1t_ref, out_ref, nd_ref):
        i = pl.program_id(0)

        @pl.when(i == 0)
        def _():
            out_ref[...] = jnp.zeros_like(out_ref)

        m_in, _ = _edge_features(xsd_ref, ea_ref)
        nd_ref[...] = m_in[10:12]
        h0 = jnp.dot(w1t_ref[...], m_in, preferred_element_type=jnp.float32)
        out_ref[:, 0:1] += jnp.sum(h0, axis=1, keepdims=True)
        out_ref[:, 1:2] += jnp.sum(h0 * h0, axis=1, keepdims=True)

    return pl.pallas_call(
        body,
        grid=(T_GRID,),
        in_specs=[
            pl.BlockSpec((6, 1, TB), lambda i: (0, 0, i)),
            pl.BlockSpec((4, TB), lambda i: (0, i)),
            pl.BlockSpec((NH, 12), lambda i: (0, 0)),
        ],
        out_specs=[
            pl.BlockSpec((NH, 128), lambda i: (0, 0)),
            pl.BlockSpec((2, TB), lambda i: (0, i)),
        ],
        out_shape=[
            jax.ShapeDtypeStruct((NH, 128), jnp.float32),
            jax.ShapeDtypeStruct((2, N_EDGES), jnp.float32),
        ],
    )(xsd, ea_t, w1t)


def _tc_mlp(xsd, ea_t, nd, w1t, scale, shift, w2ct, b2c, wx1t, bx1, wx2t):
    def body(xsd_ref, ea_ref, nd_ref, w1t_ref, scale_ref, shift_ref, w2ct_ref,
             b2c_ref, wx1t_ref, bx1_ref, wx2t_ref, out_ref):
        xs = xsd_ref[0:3, 0]
        xd = xsd_ref[3:6, 0]
        dif = xs - xd
        m_in = jnp.concatenate([xd, xs, ea_ref[...], nd_ref[...]], axis=0)
        h0 = jnp.dot(w1t_ref[...], m_in, preferred_element_type=jnp.float32)
        h1 = jax.nn.relu(h0 * scale_ref[...] + shift_ref[...])
        z = jnp.dot(w2ct_ref[...], h1,
                    preferred_element_type=jnp.float32) + b2c_ref[...]
        h2 = jax.nn.relu(z[0:NH])
        wgt = jax.nn.sigmoid(z[NH:NH + 1])
        m = h2 * wgt
        p = jax.nn.relu(
            jnp.dot(wx1t_ref[...], m,
                    preferred_element_type=jnp.float32) + bx1_ref[...])
        px = jnp.dot(wx2t_ref[...], p, preferred_element_type=jnp.float32)
        out_ref[...] = jnp.clip(dif * px, -100.0, 100.0)

    return pl.pallas_call(
        body,
        grid=(T_GRID,),
        in_specs=[
            pl.BlockSpec((6, 1, TB), lambda i: (0, 0, i)),
            pl.BlockSpec((4, TB), lambda i: (0, i)),
            pl.BlockSpec((2, TB), lambda i: (0, i)),
            pl.BlockSpec((NH, 12), lambda i: (0, 0)),
            pl.BlockSpec((NH, 1), lambda i: (0, 0)),
            pl.BlockSpec((NH, 1), lambda i: (0, 0)),
            pl.BlockSpec((NH + 1, NH), lambda i: (0, 0)),
            pl.BlockSpec((NH + 1, 1), lambda i: (0, 0)),
            pl.BlockSpec((NH, NH), lambda i: (0, 0)),
            pl.BlockSpec((NH, 1), lambda i: (0, 0)),
            pl.BlockSpec((1, NH), lambda i: (0, 0)),
        ],
        out_specs=pl.BlockSpec((3, TB), lambda i: (0, i)),
        out_shape=jax.ShapeDtypeStruct((3, E_PAD), jnp.float32),
    )(xsd, ea_t, nd, w1t, scale, shift, w2ct, b2c, wx1t, bx1, wx2t)


def _tc_final(xsd, we1t, be1, we2t, be2, we3t, be3):
    def body(xsd_ref, we1t_ref, be1_ref, we2t_ref, be2_ref, we3t_ref,
             be3_ref, out_ref):
        cat = jnp.concatenate([xsd_ref[3:6, 0], xsd_ref[0:3, 0]], axis=0)
        o1 = jax.nn.relu(
            jnp.dot(we1t_ref[...], cat,
                    preferred_element_type=jnp.float32) + be1_ref[...])
        o2 = jax.nn.relu(
            jnp.dot(we2t_ref[...], o1,
                    preferred_element_type=jnp.float32) + be2_ref[...])
        o3 = jnp.dot(we3t_ref[...], o2,
                     preferred_element_type=jnp.float32) + be3_ref[...]
        out_ref[...] = jax.nn.sigmoid(o3)

    return pl.pallas_call(
        body,
        grid=(T_GRID,),
        in_specs=[
            pl.BlockSpec((6, 1, TB), lambda i: (0, 0, i)),
            pl.BlockSpec((NH, 6), lambda i: (0, 0)),
            pl.BlockSpec((NH, 1), lambda i: (0, 0)),
            pl.BlockSpec((NH, NH), lambda i: (0, 0)),
            pl.BlockSpec((NH, 1), lambda i: (0, 0)),
            pl.BlockSpec((1, NH), lambda i: (0, 0)),
            pl.BlockSpec((1, 1), lambda i: (0, 0)),
        ],
        out_specs=pl.BlockSpec((1, TB), lambda i: (0, i)),
        out_shape=jax.ShapeDtypeStruct((1, N_EDGES), jnp.float32),
    )(xsd, we1t, be1, we2t, be2, we3t, be3)


def kernel(x, edge_index, edge_attr, W1, b1, gamma, beta, W2, b2, Wm, bm,
           Wx1, bx1, Wx2, We1, be1, We2, be2, We3, be3):
    f32 = jnp.float32
    x3 = x.T[:, None, :]
    ei3 = edge_index[:, None, :]
    ea_t = edge_attr.T
    zeros_row = jnp.zeros((NP_PAD,), f32)
    src1d = jnp.concatenate(
        [edge_index[0], jnp.full((E_PAD - N_EDGES,), N_NODES, jnp.int32)])

    for l in range(N_LAYERS):
        xsd = _sc_gather(x3, ei3)
        w1t = W1[l].T
        mom, nd = _tc_stats(xsd, ea_t, w1t)
        s1 = mom[:, 0:1] / N_EDGES
        s2 = mom[:, 1:2] / N_EDGES
        mu = s1 + b1[l][:, None]
        var = s2 - s1 * s1
        scale = gamma[l][:, None] * lax.rsqrt(var + 1e-5)
        shift = beta[l][:, None] - (mu * scale)
        w2ct = jnp.concatenate([W2[l], Wm[l]], axis=1).T
        b2c = jnp.concatenate([b2[l], bm[l]])[:, None]
        upd = _tc_mlp(xsd, ea_t, nd, w1t, scale, shift, w2ct, b2c,
                      Wx1[l].T, bx1[l][:, None], Wx2[l].T)
        partials = _sc_scatter(upd, src1d, zeros_row)
        pr = partials.reshape(2, 3, NP_PAD)
        agg = (pr[0] + pr[1])[:, :N_NODES]
        x3 = x3 + C_WEIGHT * agg[:, None, :]

    xsd = _sc_gather(x3, ei3)
    out = _tc_final(xsd, We1.T, be1[:, None], We2.T, be2[:, None],
                    We3.T, be3[None, :])
    return out.reshape(N_EDGES, 1)


# speedup vs baseline: 19.7526x; 1.2068x over previous
import functools

import jax
import jax.numpy as jnp
from jax import lax
from jax.experimental import pallas as pl
from jax.experimental.pallas import tpu as pltpu
from jax.experimental.pallas import tpu_sc as plsc

N_NODES = 100000
N_EDGES = 1600000
NH = 32
N_LAYERS = 2
C_WEIGHT = 0.001

NC = 2
NS = 16
NW = NC * NS

G_RANGES = 5
EQ = N_EDGES // G_RANGES
GK = 6400
G_NCHUNK = EQ // GK

SK = 6272
S_NCHUNK = 8
S_TILE = S_NCHUNK * SK
E_PAD = NW * S_TILE
NP_PAD = 100352
NZ = NP_PAD // NS

TB = 16000
T_GRID = N_EDGES // TB


def _psi(v):
    return jnp.sign(v) * jnp.log(jnp.abs(v) + 1.0)


def _sc_gather(x3, ei3):
    mesh = plsc.VectorSubcoreMesh(core_axis_name="c", subcore_axis_name="s")

    @functools.partial(
        pl.kernel,
        out_type=jax.ShapeDtypeStruct((6, 1, N_EDGES), jnp.float32),
        mesh=mesh,
        compiler_params=pltpu.CompilerParams(use_tc_tiling_on_sc=False, needs_layout_passes=False),
        scratch_types=[
            pltpu.VMEM((N_NODES,), jnp.float32),
            pltpu.VMEM((2, GK), jnp.int32),
            pltpu.VMEM((2, GK), jnp.float32),
            pltpu.SemaphoreType.DMA,
            pltpu.SemaphoreType.DMA,
            pltpu.SemaphoreType.DMA,
            pltpu.SemaphoreType.DMA,
        ],
    )
    def gather_k(x_hbm, ei_hbm, out_hbm, col_v, idx_v, val_v,
                 isem0, isem1, osem0, osem1):
        wid = lax.axis_index("s") * NC + lax.axis_index("c")

        @pl.when(wid < 2 * 3 * G_RANGES)
        def _():
            side = wid // (3 * G_RANGES)
            sub = wid % (3 * G_RANGES)
            coord = sub % 3
            rng = sub // 3
            out6 = side * 3 + coord
            isems = (isem0, isem1)
            osems = (osem0, osem1)

            def idx_dma(ci, b):
                return pltpu.make_async_copy(
                    ei_hbm.at[side, 0, pl.ds(rng * EQ + ci * GK, GK)],
                    idx_v.at[b], isems[b])

            def out_dma(ci, b):
                return pltpu.make_async_copy(
                    val_v.at[b],
                    out_hbm.at[out6, 0, pl.ds(rng * EQ + ci * GK, GK)],
                    osems[b])

            pltpu.sync_copy(x_hbm.at[coord, 0], col_v)
            idx_dma(0, 0).start()
            idx_dma(1, 1).start()

            @pl.loop(0, G_NCHUNK // 2)
            def _pair(t):
                for b in range(2):
                    ci = 2 * t + b
                    idx_dma(ci, b).wait()

                    @pl.when(ci >= 2)
                    def _():
                        out_dma(ci - 2, b).wait()

                    @plsc.parallel_loop(0, GK, step=16, unroll=8)
                    def _vec(j):
                        iv = idx_v[b, pl.ds(j, 16)]
                        val_v[b, pl.ds(j, 16)] = plsc.load_gather(
                            col_v, [iv])

                    out_dma(ci, b).start()

                    @pl.when(ci + 2 < G_NCHUNK)
                    def _():
                        idx_dma(ci + 2, b).start()

            out_dma(G_NCHUNK - 2, 0).wait()
            out_dma(G_NCHUNK - 1, 1).wait()

    return gather_k(x3, ei3)


def _sc_scatter(upd, src1d, zeros_row):
    mesh = plsc.VectorSubcoreMesh(core_axis_name="c", subcore_axis_name="s")

    @functools.partial(
        pl.kernel,
        out_type=jax.ShapeDtypeStruct((6, 1, NP_PAD), jnp.float32),
        mesh=mesh,
        compiler_params=pltpu.CompilerParams(use_tc_tiling_on_sc=False, needs_layout_passes=False),
        scratch_types=[
            pltpu.VMEM_SHARED((NP_PAD,), jnp.float32),
            pltpu.VMEM_SHARED((NP_PAD,), jnp.float32),
            pltpu.VMEM_SHARED((NP_PAD,), jnp.float32),
            pltpu.VMEM((2, SK), jnp.int32),
            pltpu.VMEM((2, 3, SK), jnp.float32),
            pltpu.SemaphoreType.DMA,
            pltpu.SemaphoreType.DMA,
            pltpu.SemaphoreType.DMA,
            pltpu.SemaphoreType.DMA,
            pltpu.SemaphoreType.DMA,
        ],
    )
    def scatter_k(upd_hbm, src_hbm, zrow_hbm, out_hbm, sh0, sh1, sh2,
                  idx_v, val_v, isem0, isem1, vsem0, vsem1, asem):
        cid = lax.axis_index("c")
        sid = lax.axis_index("s")
        wid = sid * NC + cid
        shared = (sh0, sh1, sh2)
        isems = (isem0, isem1)
        vsems = (vsem0, vsem1)

        def idx_dma(ci, b):
            return pltpu.make_async_copy(
                src_hbm.at[pl.ds(wid * S_TILE + ci * SK, SK)],
                idx_v.at[b], isems[b])

        def val_dma(ci, b, c):
            return pltpu.make_async_copy(
                upd_hbm.at[c, pl.ds(wid * S_TILE + ci * SK, SK)],
                val_v.at[b, c], vsems[b])

        for c in range(3):
            pltpu.sync_copy(zrow_hbm.at[pl.ds(sid * NZ, NZ)],
                            shared[c].at[pl.ds(sid * NZ, NZ)])
        plsc.subcore_barrier()

        for b in range(2):
            idx_dma(b, b).start()
            for c in range(3):
                val_dma(b, b, c).start()

        @pl.loop(0, S_NCHUNK // 2)
        def _pair(t):
            for b in range(2):
                ci = 2 * t + b
                idx_dma(ci, b).wait()
                adds = []
                for c in range(3):
                    val_dma(ci, b, c).wait()
                    adds.append(pltpu.make_async_copy(
                        val_v.at[b, c], shared[c].at[idx_v.at[b]], asem))
                for d in adds:
                    d.start(add=True)
                for d in adds:
                    d.wait()

                @pl.when(ci + 2 < S_NCHUNK)
                def _():
                    idx_dma(ci + 2, b).start()
                    for c in range(3):
                        val_dma(ci + 2, b, c).start()

        plsc.subcore_barrier()
        for c in range(3):
            pltpu.sync_copy(shared[c].at[pl.ds(sid * NZ, NZ)],
                            out_hbm.at[cid * 3 + c, 0, pl.ds(sid * NZ, NZ)])

    return scatter_k(upd, src1d, zeros_row)


def _edge_features(xsd_ref, ea_ref):
    xs = xsd_ref[0:3, 0]
    xd = xsd_ref[3:6, 0]
    dif = xs - xd
    norms = _psi(jnp.sum(dif * dif, axis=0, keepdims=True))
    dots = _psi(jnp.sum(xs * xd, axis=0, keepdims=True))
    m_in = jnp.concatenate([xd, xs, ea_ref[...], norms, dots], axis=0)
    return m_in, dif


def _tc_stats(xsd, ea_t, w1t):
    def body(xsd_ref, ea_ref, w1t_ref, out_ref, nd_ref):
        i = pl.program_id(0)

        @pl.when(i == 0)
        def _():
            out_ref[...] = jnp.zeros_like(out_ref)

        m_in, _ = _edge_features(xsd_ref, ea_ref)
        nd_ref[...] = m_in[10:12]
        h0 = jnp.dot(w1t_ref[...], m_in, preferred_element_type=jnp.float32)
        out_ref[:, 0:1] += jnp.sum(h0, axis=1, keepdims=True)
        out_ref[:, 1:2] += jnp.sum(h0 * h0, axis=1, keepdims=True)

    return pl.pallas_call(
        body,
        grid=(T_GRID,),
        in_specs=[
            pl.BlockSpec((6, 1, TB), lambda i: (0, 0, i)),
            pl.BlockSpec((4, TB), lambda i: (0, i)),
            pl.BlockSpec((NH, 12), lambda i: (0, 0)),
        ],
        out_specs=[
            pl.BlockSpec((NH, 128), lambda i: (0, 0)),
            pl.BlockSpec((2, TB), lambda i: (0, i)),
        ],
        out_shape=[
            jax.ShapeDtypeStruct((NH, 128), jnp.float32),
            jax.ShapeDtypeStruct((2, N_EDGES), jnp.float32),
        ],
    )(xsd, ea_t, w1t)


def _tc_mlp(xsd, ea_t, nd, w1t, scale, shift, w2ct, b2c, wx1t, bx1, wx2t):
    def body(xsd_ref, ea_ref, nd_ref, w1t_ref, scale_ref, shift_ref, w2ct_ref,
             b2c_ref, wx1t_ref, bx1_ref, wx2t_ref, out_ref):
        xs = xsd_ref[0:3, 0]
        xd = xsd_ref[3:6, 0]
        dif = xs - xd
        m_in = jnp.concatenate([xd, xs, ea_ref[...], nd_ref[...]], axis=0)
        h0 = jnp.dot(w1t_ref[...], m_in, preferred_element_type=jnp.float32)
        h1 = jax.nn.relu(h0 * scale_ref[...] + shift_ref[...])
        z = jnp.dot(w2ct_ref[...], h1,
                    preferred_element_type=jnp.float32) + b2c_ref[...]
        h2 = jax.nn.relu(z[0:NH])
        wgt = jax.nn.sigmoid(z[NH:NH + 1])
        m = h2 * wgt
        p = jax.nn.relu(
            jnp.dot(wx1t_ref[...], m,
                    preferred_element_type=jnp.float32) + bx1_ref[...])
        px = jnp.dot(wx2t_ref[...], p, preferred_element_type=jnp.float32)
        out_ref[...] = jnp.clip(dif * px, -100.0, 100.0)

    return pl.pallas_call(
        body,
        grid=(T_GRID,),
        in_specs=[
            pl.BlockSpec((6, 1, TB), lambda i: (0, 0, i)),
            pl.BlockSpec((4, TB), lambda i: (0, i)),
            pl.BlockSpec((2, TB), lambda i: (0, i)),
            pl.BlockSpec((NH, 12), lambda i: (0, 0)),
            pl.BlockSpec((NH, 1), lambda i: (0, 0)),
            pl.BlockSpec((NH, 1), lambda i: (0, 0)),
            pl.BlockSpec((NH + 1, NH), lambda i: (0, 0)),
            pl.BlockSpec((NH + 1, 1), lambda i: (0, 0)),
            pl.BlockSpec((NH, NH), lambda i: (0, 0)),
            pl.BlockSpec((NH, 1), lambda i: (0, 0)),
            pl.BlockSpec((1, NH), lambda i: (0, 0)),
        ],
        out_specs=pl.BlockSpec((3, TB), lambda i: (0, i)),
        out_shape=jax.ShapeDtypeStruct((3, E_PAD), jnp.float32),
    )(xsd, ea_t, nd, w1t, scale, shift, w2ct, b2c, wx1t, bx1, wx2t)


def _tc_final(xsd, we1t, be1, we2t, be2, we3t, be3):
    def body(xsd_ref, we1t_ref, be1_ref, we2t_ref, be2_ref, we3t_ref,
             be3_ref, out_ref):
        cat = jnp.concatenate([xsd_ref[3:6, 0], xsd_ref[0:3, 0]], axis=0)
        o1 = jax.nn.relu(
            jnp.dot(we1t_ref[...], cat,
                    preferred_element_type=jnp.float32) + be1_ref[...])
        o2 = jax.nn.relu(
            jnp.dot(we2t_ref[...], o1,
                    preferred_element_type=jnp.float32) + be2_ref[...])
        o3 = jnp.dot(we3t_ref[...], o2,
                     preferred_element_type=jnp.float32) + be3_ref[...]
        out_ref[...] = jax.nn.sigmoid(o3)

    return pl.pallas_call(
        body,
        grid=(T_GRID,),
        in_specs=[
            pl.BlockSpec((6, 1, TB), lambda i: (0, 0, i)),
            pl.BlockSpec((NH, 6), lambda i: (0, 0)),
            pl.BlockSpec((NH, 1), lambda i: (0, 0)),
            pl.BlockSpec((NH, NH), lambda i: (0, 0)),
            pl.BlockSpec((NH, 1), lambda i: (0, 0)),
            pl.BlockSpec((1, NH), lambda i: (0, 0)),
            pl.BlockSpec((1, 1), lambda i: (0, 0)),
        ],
        out_specs=pl.BlockSpec((1, TB), lambda i: (0, i)),
        out_shape=jax.ShapeDtypeStruct((1, N_EDGES), jnp.float32),
    )(xsd, we1t, be1, we2t, be2, we3t, be3)


def kernel(x, edge_index, edge_attr, W1, b1, gamma, beta, W2, b2, Wm, bm,
           Wx1, bx1, Wx2, We1, be1, We2, be2, We3, be3):
    f32 = jnp.float32
    x3 = x.T[:, None, :]
    ei3 = edge_index[:, None, :]
    ea_t = edge_attr.T
    zeros_row = jnp.zeros((NP_PAD,), f32)
    src1d = jnp.concatenate(
        [edge_index[0], jnp.full((E_PAD - N_EDGES,), N_NODES, jnp.int32)])

    for l in range(N_LAYERS):
        xsd = _sc_gather(x3, ei3)
        w1t = W1[l].T
        mom, nd = _tc_stats(xsd, ea_t, w1t)
        s1 = mom[:, 0:1] / N_EDGES
        s2 = mom[:, 1:2] / N_EDGES
        mu = s1 + b1[l][:, None]
        var = s2 - s1 * s1
        scale = gamma[l][:, None] * lax.rsqrt(var + 1e-5)
        shift = beta[l][:, None] - (mu * scale)
        w2ct = jnp.concatenate([W2[l], Wm[l]], axis=1).T
        b2c = jnp.concatenate([b2[l], bm[l]])[:, None]
        upd = _tc_mlp(xsd, ea_t, nd, w1t, scale, shift, w2ct, b2c,
                      Wx1[l].T, bx1[l][:, None], Wx2[l].T)
        partials = _sc_scatter(upd, src1d, zeros_row)
        pr = partials.reshape(2, 3, NP_PAD)
        agg = (pr[0] + pr[1])[:, :N_NODES]
        x3 = x3 + C_WEIGHT * agg[:, None, :]

    xsd = _sc_gather(x3, ei3)
    out = _tc_final(xsd, We1.T, be1[:, None], We2.T, be2[:, None],
                    We3.T, be3[None, :])
    return out.reshape(N_EDGES, 1)


# speedup vs baseline: 19.8394x; 1.0044x over previous
import functools

import jax
import jax.numpy as jnp
from jax import lax
from jax.experimental import pallas as pl
from jax.experimental.pallas import tpu as pltpu
from jax.experimental.pallas import tpu_sc as plsc

N_NODES = 100000
N_EDGES = 1600000
NH = 32
N_LAYERS = 2
C_WEIGHT = 0.001

NC = 2
NS = 16
NW = NC * NS

G_RANGES = 5
EQ = N_EDGES // G_RANGES
GK = 6400
G_NCHUNK = EQ // GK

SK = 6272
S_NCHUNK = 8
S_TILE = S_NCHUNK * SK
E_PAD = NW * S_TILE
NP_PAD = 100352
NZ = NP_PAD // NS

TB = 16000
T_GRID = N_EDGES // TB


def _psi(v):
    return jnp.sign(v) * jnp.log(jnp.abs(v) + 1.0)


def _sc_gather(x3, ei3):
    mesh = plsc.VectorSubcoreMesh(core_axis_name="c", subcore_axis_name="s")

    @functools.partial(
        pl.kernel,
        out_type=jax.ShapeDtypeStruct((6, 1, N_EDGES), jnp.float32),
        mesh=mesh,
        compiler_params=pltpu.CompilerParams(use_tc_tiling_on_sc=False, needs_layout_passes=False),
        scratch_types=[
            pltpu.VMEM((N_NODES,), jnp.float32),
            pltpu.VMEM((2, GK), jnp.int32),
            pltpu.VMEM((2, GK), jnp.float32),
            pltpu.SemaphoreType.DMA,
            pltpu.SemaphoreType.DMA,
            pltpu.SemaphoreType.DMA,
            pltpu.SemaphoreType.DMA,
        ],
    )
    def gather_k(x_hbm, ei_hbm, out_hbm, col_v, idx_v, val_v,
                 isem0, isem1, osem0, osem1):
        wid = lax.axis_index("s") * NC + lax.axis_index("c")

        @pl.when(wid < 2 * 3 * G_RANGES)
        def _():
            side = wid // (3 * G_RANGES)
            sub = wid % (3 * G_RANGES)
            coord = sub % 3
            rng = sub // 3
            out6 = side * 3 + coord
            isems = (isem0, isem1)
            osems = (osem0, osem1)

            def idx_dma(ci, b):
                return pltpu.make_async_copy(
                    ei_hbm.at[side, 0, pl.ds(rng * EQ + ci * GK, GK)],
                    idx_v.at[b], isems[b])

            def out_dma(ci, b):
                return pltpu.make_async_copy(
                    val_v.at[b],
                    out_hbm.at[out6, 0, pl.ds(rng * EQ + ci * GK, GK)],
                    osems[b])

            pltpu.sync_copy(x_hbm.at[coord, 0], col_v)
            idx_dma(0, 0).start()
            idx_dma(1, 1).start()

            @pl.loop(0, G_NCHUNK // 2)
            def _pair(t):
                for b in range(2):
                    ci = 2 * t + b
                    idx_dma(ci, b).wait()

                    @pl.when(ci >= 2)
                    def _():
                        out_dma(ci - 2, b).wait()

                    @plsc.parallel_loop(0, GK, step=16, unroll=8)
                    def _vec(j):
                        iv = idx_v[b, pl.ds(j, 16)]
                        val_v[b, pl.ds(j, 16)] = plsc.load_gather(
                            col_v, [iv])

                    out_dma(ci, b).start()

                    @pl.when(ci + 2 < G_NCHUNK)
                    def _():
                        idx_dma(ci + 2, b).start()

            out_dma(G_NCHUNK - 2, 0).wait()
            out_dma(G_NCHUNK - 1, 1).wait()

    return gather_k(x3, ei3)


def _sc_scatter(upd, src1d, zeros_row):
    mesh = plsc.VectorSubcoreMesh(core_axis_name="c", subcore_axis_name="s")

    @functools.partial(
        pl.kernel,
        out_type=jax.ShapeDtypeStruct((6, 1, NP_PAD), jnp.float32),
        mesh=mesh,
        compiler_params=pltpu.CompilerParams(use_tc_tiling_on_sc=False, needs_layout_passes=False),
        scratch_types=[
            pltpu.VMEM_SHARED((NP_PAD,), jnp.float32),
            pltpu.VMEM_SHARED((NP_PAD,), jnp.float32),
            pltpu.VMEM_SHARED((NP_PAD,), jnp.float32),
            pltpu.VMEM((2, SK), jnp.int32),
            pltpu.VMEM((2, 3, SK), jnp.float32),
            pltpu.SemaphoreType.DMA,
            pltpu.SemaphoreType.DMA,
            pltpu.SemaphoreType.DMA,
            pltpu.SemaphoreType.DMA,
            pltpu.SemaphoreType.DMA,
        ],
    )
    def scatter_k(upd_hbm, src_hbm, zrow_hbm, out_hbm, sh0, sh1, sh2,
                  idx_v, val_v, isem0, isem1, vsem0, vsem1, asem):
        cid = lax.axis_index("c")
        sid = lax.axis_index("s")
        wid = sid * NC + cid
        shared = (sh0, sh1, sh2)
        isems = (isem0, isem1)
        vsems = (vsem0, vsem1)

        def idx_dma(ci, b):
            return pltpu.make_async_copy(
                src_hbm.at[pl.ds(wid * S_TILE + ci * SK, SK)],
                idx_v.at[b], isems[b])

        def val_dma(ci, b, c):
            return pltpu.make_async_copy(
                upd_hbm.at[c, pl.ds(wid * S_TILE + ci * SK, SK)],
                val_v.at[b, c], vsems[b])

        for c in range(3):
            pltpu.sync_copy(zrow_hbm.at[pl.ds(sid * NZ, NZ)],
                            shared[c].at[pl.ds(sid * NZ, NZ)])
        plsc.subcore_barrier()

        for b in range(2):
            idx_dma(b, b).start()
            for c in range(3):
                val_dma(b, b, c).start()

        @pl.loop(0, S_NCHUNK // 2)
        def _pair(t):
            for b in range(2):
                ci = 2 * t + b
                idx_dma(ci, b).wait()
                adds = []
                for c in range(3):
                    val_dma(ci, b, c).wait()
                    adds.append(pltpu.make_async_copy(
                        val_v.at[b, c], shared[c].at[idx_v.at[b]], asem))
                for d in adds:
                    d.start(add=True)
                for d in adds:
                    d.wait()

                @pl.when(ci + 2 < S_NCHUNK)
                def _():
                    idx_dma(ci + 2, b).start()
                    for c in range(3):
                        val_dma(ci + 2, b, c).start()

        plsc.subcore_barrier()
        for c in range(3):
            pltpu.sync_copy(shared[c].at[pl.ds(sid * NZ, NZ)],
                            out_hbm.at[cid * 3 + c, 0, pl.ds(sid * NZ, NZ)])

    return scatter_k(upd, src1d, zeros_row)


def _edge_features(xsd_ref, ea_ref):
    xs = xsd_ref[0:3, 0]
    xd = xsd_ref[3:6, 0]
    dif = xs - xd
    norms = _psi(jnp.sum(dif * dif, axis=0, keepdims=True))
    dots = _psi(jnp.sum(xs * xd, axis=0, keepdims=True))
    m_in = jnp.concatenate([xd, xs, ea_ref[...], norms, dots], axis=0)
    return m_in, dif


def _tc_stats(xsd, ea_t, w1t):
    def body(xsd_ref, ea_ref, w1t_ref, out_ref, nd_ref):
        i = pl.program_id(0)

        @pl.when(i == 0)
        def _():
            out_ref[...] = jnp.zeros_like(out_ref)

        m_in, _ = _edge_features(xsd_ref, ea_ref)
        nd_ref[...] = m_in[10:12]
        h0 = jnp.dot(w1t_ref[...], m_in, preferred_element_type=jnp.float32)
        out_ref[:, 0:1] += jnp.sum(h0, axis=1, keepdims=True)
        out_ref[:, 1:2] += jnp.sum(h0 * h0, axis=1, keepdims=True)

    return pl.pallas_call(
        body,
        grid=(T_GRID,),
        in_specs=[
            pl.BlockSpec((6, 1, TB), lambda i: (0, 0, i)),
            pl.BlockSpec((4, TB), lambda i: (0, i)),
            pl.BlockSpec((NH, 12), lambda i: (0, 0)),
        ],
        out_specs=[
            pl.BlockSpec((NH, 128), lambda i: (0, 0)),
            pl.BlockSpec((2, TB), lambda i: (0, i)),
        ],
        out_shape=[
            jax.ShapeDtypeStruct((NH, 128), jnp.float32),
            jax.ShapeDtypeStruct((2, N_EDGES), jnp.float32),
        ],
    )(xsd, ea_t, w1t)


def _tc_mlp(xsd, ea_t, nd, w1t, scale, shift, w2ct, b2c, wx1t, bx1, wx2t):
    def body(xsd_ref, ea_ref, nd_ref, w1t_ref, scale_ref, shift_ref, w2ct_ref,
             b2c_ref, wx1t_ref, bx1_ref, wx2t_ref, out_ref):
        xs = xsd_ref[0:3, 0]
        xd = xsd_ref[3:6, 0]
        dif = xs - xd
        m_in = jnp.concatenate([xd, xs, ea_ref[...], nd_ref[...]], axis=0)
        h0 = jnp.dot(w1t_ref[...], m_in, preferred_element_type=jnp.float32)
        h1 = jax.nn.relu(h0 * scale_ref[...] + shift_ref[...])
        z = jnp.dot(w2ct_ref[...], h1,
                    preferred_element_type=jnp.float32) + b2c_ref[...]
        h2 = jax.nn.relu(z[0:NH])
        wgt = jax.nn.sigmoid(z[NH:NH + 1])
        m = h2 * wgt
        p = jax.nn.relu(
            jnp.dot(wx1t_ref[...], m,
                    preferred_element_type=jnp.float32) + bx1_ref[...])
        px = jnp.dot(wx2t_ref[...], p, preferred_element_type=jnp.float32)
        out_ref[...] = jnp.clip(dif * px, -100.0, 100.0)

    return pl.pallas_call(
        body,
        grid=(T_GRID,),
        in_specs=[
            pl.BlockSpec((6, 1, TB), lambda i: (0, 0, i)),
            pl.BlockSpec((4, TB), lambda i: (0, i)),
            pl.BlockSpec((2, TB), lambda i: (0, i)),
            pl.BlockSpec((NH, 12), lambda i: (0, 0)),
            pl.BlockSpec((NH, 1), lambda i: (0, 0)),
            pl.BlockSpec((NH, 1), lambda i: (0, 0)),
            pl.BlockSpec((NH + 1, NH), lambda i: (0, 0)),
            pl.BlockSpec((NH + 1, 1), lambda i: (0, 0)),
            pl.BlockSpec((NH, NH), lambda i: (0, 0)),
            pl.BlockSpec((NH, 1), lambda i: (0, 0)),
            pl.BlockSpec((1, NH), lambda i: (0, 0)),
        ],
        out_specs=pl.BlockSpec((3, TB), lambda i: (0, i)),
        out_shape=jax.ShapeDtypeStruct((3, E_PAD), jnp.float32),
    )(xsd, ea_t, nd, w1t, scale, shift, w2ct, b2c, wx1t, bx1, wx2t)


XB = 12544


def _tc_xupdate(x3, partials):
    def body(x_ref, p_ref, out_ref):
        out_ref[...] = x_ref[...] + C_WEIGHT * (
            p_ref[0:3] + p_ref[3:6])

    return pl.pallas_call(
        body,
        grid=(NP_PAD // XB,),
        in_specs=[
            pl.BlockSpec((3, 1, XB), lambda i: (0, 0, i)),
            pl.BlockSpec((6, 1, XB), lambda i: (0, 0, i)),
        ],
        out_specs=pl.BlockSpec((3, 1, XB), lambda i: (0, 0, i)),
        out_shape=jax.ShapeDtypeStruct((3, 1, N_NODES), jnp.float32),
    )(x3, partials)


def _tc_final(xsd, we1t, be1, we2t, be2, we3t, be3):
    def body(xsd_ref, we1t_ref, be1_ref, we2t_ref, be2_ref, we3t_ref,
             be3_ref, out_ref):
        cat = jnp.concatenate([xsd_ref[3:6, 0], xsd_ref[0:3, 0]], axis=0)
        o1 = jax.nn.relu(
            jnp.dot(we1t_ref[...], cat,
                    preferred_element_type=jnp.float32) + be1_ref[...])
        o2 = jax.nn.relu(
            jnp.dot(we2t_ref[...], o1,
                    preferred_element_type=jnp.float32) + be2_ref[...])
        o3 = jnp.dot(we3t_ref[...], o2,
                     preferred_element_type=jnp.float32) + be3_ref[...]
        out_ref[...] = jax.nn.sigmoid(o3)

    return pl.pallas_call(
        body,
        grid=(T_GRID,),
        in_specs=[
            pl.BlockSpec((6, 1, TB), lambda i: (0, 0, i)),
            pl.BlockSpec((NH, 6), lambda i: (0, 0)),
            pl.BlockSpec((NH, 1), lambda i: (0, 0)),
            pl.BlockSpec((NH, NH), lambda i: (0, 0)),
            pl.BlockSpec((NH, 1), lambda i: (0, 0)),
            pl.BlockSpec((1, NH), lambda i: (0, 0)),
            pl.BlockSpec((1, 1), lambda i: (0, 0)),
        ],
        out_specs=pl.BlockSpec((1, TB), lambda i: (0, i)),
        out_shape=jax.ShapeDtypeStruct((1, N_EDGES), jnp.float32),
    )(xsd, we1t, be1, we2t, be2, we3t, be3)


def kernel(x, edge_index, edge_attr, W1, b1, gamma, beta, W2, b2, Wm, bm,
           Wx1, bx1, Wx2, We1, be1, We2, be2, We3, be3):
    f32 = jnp.float32
    x3 = x.T[:, None, :]
    ei3 = edge_index[:, None, :]
    ea_t = edge_attr.T
    zeros_row = jnp.zeros((NP_PAD,), f32)
    src1d = jnp.concatenate(
        [edge_index[0], jnp.full((E_PAD - N_EDGES,), N_NODES, jnp.int32)])

    for l in range(N_LAYERS):
        xsd = _sc_gather(x3, ei3)
        w1t = W1[l].T
        mom, nd = _tc_stats(xsd, ea_t, w1t)
        s1 = mom[:, 0:1] / N_EDGES
        s2 = mom[:, 1:2] / N_EDGES
        mu = s1 + b1[l][:, None]
        var = s2 - s1 * s1
        scale = gamma[l][:, None] * lax.rsqrt(var + 1e-5)
        shift = beta[l][:, None] - (mu * scale)
        w2ct = jnp.concatenate([W2[l], Wm[l]], axis=1).T
        b2c = jnp.concatenate([b2[l], bm[l]])[:, None]
        upd = _tc_mlp(xsd, ea_t, nd, w1t, scale, shift, w2ct, b2c,
                      Wx1[l].T, bx1[l][:, None], Wx2[l].T)
        partials = _sc_scatter(upd, src1d, zeros_row)
        x3 = _tc_xupdate(x3, partials)

    xsd = _sc_gather(x3, ei3)
    out = _tc_final(xsd, We1.T, be1[:, None], We2.T, be2[:, None],
                    We3.T, be3[None, :])
    return out.reshape(N_EDGES, 1)


# speedup vs baseline: 33.4840x; 1.6877x over previous
import functools

import jax
import jax.numpy as jnp
from jax import lax
from jax.experimental import pallas as pl
from jax.experimental.pallas import tpu as pltpu
from jax.experimental.pallas import tpu_sc as plsc

N_NODES = 100000
N_EDGES = 1600000
NH = 32
N_LAYERS = 2
C_WEIGHT = 0.001

NC = 2
NS = 16
NW = NC * NS

G_RANGES = 5
EQ = N_EDGES // G_RANGES
GK = 6400
G_NCHUNK = EQ // GK

SK = 6272
S_NCHUNK = 8
S_TILE = S_NCHUNK * SK
E_PAD = NW * S_TILE
NP_PAD = 100352
NZ = NP_PAD // NS

TB = 16000
T_GRID = N_EDGES // TB
TB2 = 16384
T_GRID2 = E_PAD // TB2


def _psi(v):
    return jnp.sign(v) * jnp.log(jnp.abs(v) + 1.0)


def _sc_gather(x3, ei3):
    mesh = plsc.VectorSubcoreMesh(core_axis_name="c", subcore_axis_name="s")

    @functools.partial(
        pl.kernel,
        out_type=jax.ShapeDtypeStruct((6, 1, N_EDGES), jnp.float32),
        mesh=mesh,
        compiler_params=pltpu.CompilerParams(use_tc_tiling_on_sc=False, needs_layout_passes=False),
        scratch_types=[
            pltpu.VMEM((N_NODES,), jnp.float32),
            pltpu.VMEM((2, GK), jnp.int32),
            pltpu.VMEM((2, GK), jnp.float32),
            pltpu.SemaphoreType.DMA,
            pltpu.SemaphoreType.DMA,
            pltpu.SemaphoreType.DMA,
            pltpu.SemaphoreType.DMA,
        ],
    )
    def gather_k(x_hbm, ei_hbm, out_hbm, col_v, idx_v, val_v,
                 isem0, isem1, osem0, osem1):
        wid = lax.axis_index("s") * NC + lax.axis_index("c")

        @pl.when(wid < 2 * 3 * G_RANGES)
        def _():
            side = wid // (3 * G_RANGES)
            sub = wid % (3 * G_RANGES)
            coord = sub % 3
            rng = sub // 3
            out6 = side * 3 + coord
            isems = (isem0, isem1)
            osems = (osem0, osem1)

            def idx_dma(ci, b):
                return pltpu.make_async_copy(
                    ei_hbm.at[side, 0, pl.ds(rng * EQ + ci * GK, GK)],
                    idx_v.at[b], isems[b])

            def out_dma(ci, b):
                return pltpu.make_async_copy(
                    val_v.at[b],
                    out_hbm.at[out6, 0, pl.ds(rng * EQ + ci * GK, GK)],
                    osems[b])

            pltpu.sync_copy(x_hbm.at[coord, 0], col_v)
            idx_dma(0, 0).start()
            idx_dma(1, 1).start()

            @pl.loop(0, G_NCHUNK // 2)
            def _pair(t):
                for b in range(2):
                    ci = 2 * t + b
                    idx_dma(ci, b).wait()

                    @pl.when(ci >= 2)
                    def _():
                        out_dma(ci - 2, b).wait()

                    @plsc.parallel_loop(0, GK, step=16, unroll=8)
                    def _vec(j):
                        iv = idx_v[b, pl.ds(j, 16)]
                        val_v[b, pl.ds(j, 16)] = plsc.load_gather(
                            col_v, [iv])

                    out_dma(ci, b).start()

                    @pl.when(ci + 2 < G_NCHUNK)
                    def _():
                        idx_dma(ci + 2, b).start()

            out_dma(G_NCHUNK - 2, 0).wait()
            out_dma(G_NCHUNK - 1, 1).wait()

    return gather_k(x3, ei3)


def _sc_scatter(upd0, upd1, upd2, src1d, zeros_row):
    mesh = plsc.VectorSubcoreMesh(core_axis_name="c", subcore_axis_name="s")

    @functools.partial(
        pl.kernel,
        out_type=jax.ShapeDtypeStruct((6, 1, NP_PAD), jnp.float32),
        mesh=mesh,
        compiler_params=pltpu.CompilerParams(use_tc_tiling_on_sc=False, needs_layout_passes=False),
        scratch_types=[
            pltpu.VMEM_SHARED((NP_PAD,), jnp.float32),
            pltpu.VMEM_SHARED((NP_PAD,), jnp.float32),
            pltpu.VMEM_SHARED((NP_PAD,), jnp.float32),
            pltpu.VMEM((2, SK), jnp.int32),
            pltpu.VMEM((2, 3, SK), jnp.float32),
            pltpu.SemaphoreType.DMA,
            pltpu.SemaphoreType.DMA,
            pltpu.SemaphoreType.DMA,
            pltpu.SemaphoreType.DMA,
            pltpu.SemaphoreType.DMA,
        ],
    )
    def scatter_k(upd0_hbm, upd1_hbm, upd2_hbm, src_hbm, zrow_hbm, out_hbm,
                  sh0, sh1, sh2, idx_v, val_v, isem0, isem1, vsem0, vsem1,
                  asem):
        cid = lax.axis_index("c")
        sid = lax.axis_index("s")
        wid = sid * NC + cid
        shared = (sh0, sh1, sh2)
        upds = (upd0_hbm, upd1_hbm, upd2_hbm)
        isems = (isem0, isem1)
        vsems = (vsem0, vsem1)

        def idx_dma(ci, b):
            return pltpu.make_async_copy(
                src_hbm.at[pl.ds(wid * S_TILE + ci * SK, SK)],
                idx_v.at[b], isems[b])

        def val_dma(ci, b, c):
            return pltpu.make_async_copy(
                upds[c].at[pl.ds(wid * S_TILE + ci * SK, SK)],
                val_v.at[b, c], vsems[b])

        for c in range(3):
            pltpu.sync_copy(zrow_hbm.at[pl.ds(sid * NZ, NZ)],
                            shared[c].at[pl.ds(sid * NZ, NZ)])
        plsc.subcore_barrier()

        for b in range(2):
            idx_dma(b, b).start()
            for c in range(3):
                val_dma(b, b, c).start()

        @pl.loop(0, S_NCHUNK // 2)
        def _pair(t):
            for b in range(2):
                ci = 2 * t + b
                idx_dma(ci, b).wait()
                adds = []
                for c in range(3):
                    val_dma(ci, b, c).wait()
                    adds.append(pltpu.make_async_copy(
                        val_v.at[b, c], shared[c].at[idx_v.at[b]], asem))
                for d in adds:
                    d.start(add=True)
                for d in adds:
                    d.wait()

                @pl.when(ci + 2 < S_NCHUNK)
                def _():
                    idx_dma(ci + 2, b).start()
                    for c in range(3):
                        val_dma(ci + 2, b, c).start()

        plsc.subcore_barrier()
        for c in range(3):
            pltpu.sync_copy(shared[c].at[pl.ds(sid * NZ, NZ)],
                            out_hbm.at[cid * 3 + c, 0, pl.ds(sid * NZ, NZ)])

    return scatter_k(upd0, upd1, upd2, src1d, zeros_row)


def _edge_features(xsd_ref, ea_ref):
    xs = xsd_ref[0:3, 0]
    xd = xsd_ref[3:6, 0]
    dif = xs - xd
    norms = _psi(jnp.sum(dif * dif, axis=0, keepdims=True))
    dots = _psi(jnp.sum(xs * xd, axis=0, keepdims=True))
    m_in = jnp.concatenate([xd, xs, ea_ref[...], norms, dots], axis=0)
    return m_in, dif


def _tc_stats(xsd, ea_t, w1t):
    def body(xsd_ref, ea_ref, w1t_ref, out_ref, nd_ref):
        i = pl.program_id(0)

        @pl.when(i == 0)
        def _():
            out_ref[...] = jnp.zeros_like(out_ref)

        m_in, _ = _edge_features(xsd_ref, ea_ref)
        nd_ref[...] = m_in[10:12]
        h0 = jnp.dot(w1t_ref[...], m_in, preferred_element_type=jnp.float32)
        out_ref[:, 0:1] += jnp.sum(h0, axis=1, keepdims=True)
        out_ref[:, 1:2] += jnp.sum(h0 * h0, axis=1, keepdims=True)

    return pl.pallas_call(
        body,
        grid=(T_GRID,),
        in_specs=[
            pl.BlockSpec((6, 1, TB), lambda i: (0, 0, i)),
            pl.BlockSpec((4, TB), lambda i: (0, i)),
            pl.BlockSpec((NH, 12), lambda i: (0, 0)),
        ],
        out_specs=[
            pl.BlockSpec((NH, 128), lambda i: (0, 0)),
            pl.BlockSpec((2, TB), lambda i: (0, i)),
        ],
        out_shape=[
            jax.ShapeDtypeStruct((NH, 128), jnp.float32),
            jax.ShapeDtypeStruct((2, N_EDGES), jnp.float32),
        ],
    )(xsd, ea_t, w1t)


def _tc_mlp(xsd, ea_t, nd, w1t, scale, shift, w2ct, b2c, wx1t, bx1, wx2t):
    def body(xsd_ref, ea_ref, nd_ref, w1t_ref, scale_ref, shift_ref, w2ct_ref,
             b2c_ref, wx1t_ref, bx1_ref, wx2t_ref, o0_ref, o1_ref, o2_ref):
        xs = xsd_ref[0:3, 0]
        xd = xsd_ref[3:6, 0]
        dif = xs - xd
        m_in = jnp.concatenate([xd, xs, ea_ref[...], nd_ref[...]], axis=0)
        h0 = jnp.dot(w1t_ref[...], m_in, preferred_element_type=jnp.float32)
        h1 = jax.nn.relu(h0 * scale_ref[...] + shift_ref[...])
        z = jnp.dot(w2ct_ref[...], h1,
                    preferred_element_type=jnp.float32) + b2c_ref[...]
        h2 = jax.nn.relu(z[0:NH])
        wgt = jax.nn.sigmoid(z[NH:NH + 1])
        m = h2 * wgt
        p = jax.nn.relu(
            jnp.dot(wx1t_ref[...], m,
                    preferred_element_type=jnp.float32) + bx1_ref[...])
        px = jnp.dot(wx2t_ref[...], p, preferred_element_type=jnp.float32)
        u = jnp.clip(dif * px, -100.0, 100.0)
        o0_ref[...] = u[0]
        o1_ref[...] = u[1]
        o2_ref[...] = u[2]

    return pl.pallas_call(
        body,
        grid=(T_GRID2,),
        in_specs=[
            pl.BlockSpec((6, 1, TB2), lambda i: (0, 0, i)),
            pl.BlockSpec((4, TB2), lambda i: (0, i)),
            pl.BlockSpec((2, TB2), lambda i: (0, i)),
            pl.BlockSpec((NH, 12), lambda i: (0, 0)),
            pl.BlockSpec((NH, 1), lambda i: (0, 0)),
            pl.BlockSpec((NH, 1), lambda i: (0, 0)),
            pl.BlockSpec((NH + 1, NH), lambda i: (0, 0)),
            pl.BlockSpec((NH + 1, 1), lambda i: (0, 0)),
            pl.BlockSpec((NH, NH), lambda i: (0, 0)),
            pl.BlockSpec((NH, 1), lambda i: (0, 0)),
            pl.BlockSpec((1, NH), lambda i: (0, 0)),
        ],
        out_specs=[
            pl.BlockSpec((TB2,), lambda i: (i,)),
            pl.BlockSpec((TB2,), lambda i: (i,)),
            pl.BlockSpec((TB2,), lambda i: (i,)),
        ],
        out_shape=[
            jax.ShapeDtypeStruct((E_PAD,), jnp.float32),
            jax.ShapeDtypeStruct((E_PAD,), jnp.float32),
            jax.ShapeDtypeStruct((E_PAD,), jnp.float32),
        ],
    )(xsd, ea_t, nd, w1t, scale, shift, w2ct, b2c, wx1t, bx1, wx2t)


XB = 12544


def _tc_xupdate(x3, partials):
    def body(x_ref, p_ref, out_ref):
        out_ref[...] = x_ref[...] + C_WEIGHT * (
            p_ref[0:3] + p_ref[3:6])

    return pl.pallas_call(
        body,
        grid=(NP_PAD // XB,),
        in_specs=[
            pl.BlockSpec((3, 1, XB), lambda i: (0, 0, i)),
            pl.BlockSpec((6, 1, XB), lambda i: (0, 0, i)),
        ],
        out_specs=pl.BlockSpec((3, 1, XB), lambda i: (0, 0, i)),
        out_shape=jax.ShapeDtypeStruct((3, 1, N_NODES), jnp.float32),
    )(x3, partials)


def _tc_final(xsd, we1t, be1, we2t, be2, we3t, be3):
    def body(xsd_ref, we1t_ref, be1_ref, we2t_ref, be2_ref, we3t_ref,
             be3_ref, out_ref):
        cat = jnp.concatenate([xsd_ref[3:6, 0], xsd_ref[0:3, 0]], axis=0)
        o1 = jax.nn.relu(
            jnp.dot(we1t_ref[...], cat,
                    preferred_element_type=jnp.float32) + be1_ref[...])
        o2 = jax.nn.relu(
            jnp.dot(we2t_ref[...], o1,
                    preferred_element_type=jnp.float32) + be2_ref[...])
        o3 = jnp.dot(we3t_ref[...], o2,
                     preferred_element_type=jnp.float32) + be3_ref[...]
        out_ref[...] = jax.nn.sigmoid(o3)

    return pl.pallas_call(
        body,
        grid=(T_GRID,),
        in_specs=[
            pl.BlockSpec((6, 1, TB), lambda i: (0, 0, i)),
            pl.BlockSpec((NH, 6), lambda i: (0, 0)),
            pl.BlockSpec((NH, 1), lambda i: (0, 0)),
            pl.BlockSpec((NH, NH), lambda i: (0, 0)),
            pl.BlockSpec((NH, 1), lambda i: (0, 0)),
            pl.BlockSpec((1, NH), lambda i: (0, 0)),
            pl.BlockSpec((1, 1), lambda i: (0, 0)),
        ],
        out_specs=pl.BlockSpec((1, TB), lambda i: (0, i)),
        out_shape=jax.ShapeDtypeStruct((1, N_EDGES), jnp.float32),
    )(xsd, we1t, be1, we2t, be2, we3t, be3)


def kernel(x, edge_index, edge_attr, W1, b1, gamma, beta, W2, b2, Wm, bm,
           Wx1, bx1, Wx2, We1, be1, We2, be2, We3, be3):
    f32 = jnp.float32
    x3 = x.T[:, None, :]
    ei3 = edge_index[:, None, :]
    ea_t = edge_attr.T
    zeros_row = jnp.zeros((NP_PAD,), f32)
    src1d = jnp.concatenate(
        [edge_index[0], jnp.full((E_PAD - N_EDGES,), N_NODES, jnp.int32)])

    for l in range(N_LAYERS):
        xsd = _sc_gather(x3, ei3)
        w1t = W1[l].T
        mom, nd = _tc_stats(xsd, ea_t, w1t)
        s1 = mom[:, 0:1] / N_EDGES
        s2 = mom[:, 1:2] / N_EDGES
        mu = s1 + b1[l][:, None]
        var = s2 - s1 * s1
        scale = gamma[l][:, None] * lax.rsqrt(var + 1e-5)
        shift = beta[l][:, None] - (mu * scale)
        w2ct = jnp.concatenate([W2[l], Wm[l]], axis=1).T
        b2c = jnp.concatenate([b2[l], bm[l]])[:, None]
        upd0, upd1, upd2 = _tc_mlp(xsd, ea_t, nd, w1t, scale, shift, w2ct,
                                   b2c, Wx1[l].T, bx1[l][:, None], Wx2[l].T)
        partials = _sc_scatter(upd0, upd1, upd2, src1d, zeros_row)
        x3 = _tc_xupdate(x3, partials)

    xsd = _sc_gather(x3, ei3)
    out = _tc_final(xsd, We1.T, be1[:, None], We2.T, be2[:, None],
                    We3.T, be3[None, :])
    return out.reshape(N_EDGES, 1)


# speedup vs baseline: 34.1939x; 1.0212x over previous
import functools

import jax
import jax.numpy as jnp
from jax import lax
from jax.experimental import pallas as pl
from jax.experimental.pallas import tpu as pltpu
from jax.experimental.pallas import tpu_sc as plsc

N_NODES = 100000
N_EDGES = 1600000
NH = 32
N_LAYERS = 2
C_WEIGHT = 0.001

NC = 2
NS = 16
NW = NC * NS

G_RANGES = 5
EQ = N_EDGES // G_RANGES
GK = 6400
G_NCHUNK = EQ // GK

SK = 6272
S_NCHUNK = 8
S_TILE = S_NCHUNK * SK
E_PAD = NW * S_TILE
NP_PAD = 100352
NZ = NP_PAD // NS

TB = 32000
T_GRID = N_EDGES // TB
TB2 = 32768
T_GRID2 = E_PAD // TB2


def _psi(v):
    return jnp.sign(v) * jnp.log(jnp.abs(v) + 1.0)


def _sc_gather(x3, ei3):
    mesh = plsc.VectorSubcoreMesh(core_axis_name="c", subcore_axis_name="s")

    @functools.partial(
        pl.kernel,
        out_type=jax.ShapeDtypeStruct((6, 1, N_EDGES), jnp.float32),
        mesh=mesh,
        compiler_params=pltpu.CompilerParams(use_tc_tiling_on_sc=False, needs_layout_passes=False),
        scratch_types=[
            pltpu.VMEM((N_NODES,), jnp.float32),
            pltpu.VMEM((2, GK), jnp.int32),
            pltpu.VMEM((2, GK), jnp.float32),
            pltpu.SemaphoreType.DMA,
            pltpu.SemaphoreType.DMA,
            pltpu.SemaphoreType.DMA,
            pltpu.SemaphoreType.DMA,
        ],
    )
    def gather_k(x_hbm, ei_hbm, out_hbm, col_v, idx_v, val_v,
                 isem0, isem1, osem0, osem1):
        wid = lax.axis_index("s") * NC + lax.axis_index("c")

        @pl.when(wid < 2 * 3 * G_RANGES)
        def _():
            side = wid // (3 * G_RANGES)
            sub = wid % (3 * G_RANGES)
            coord = sub % 3
            rng = sub // 3
            out6 = side * 3 + coord
            isems = (isem0, isem1)
            osems = (osem0, osem1)

            def idx_dma(ci, b):
                return pltpu.make_async_copy(
                    ei_hbm.at[side, 0, pl.ds(rng * EQ + ci * GK, GK)],
                    idx_v.at[b], isems[b])

            def out_dma(ci, b):
                return pltpu.make_async_copy(
                    val_v.at[b],
                    out_hbm.at[out6, 0, pl.ds(rng * EQ + ci * GK, GK)],
                    osems[b])

            pltpu.sync_copy(x_hbm.at[coord, 0], col_v)
            idx_dma(0, 0).start()
            idx_dma(1, 1).start()

            @pl.loop(0, G_NCHUNK // 2)
            def _pair(t):
                for b in range(2):
                    ci = 2 * t + b
                    idx_dma(ci, b).wait()

                    @pl.when(ci >= 2)
                    def _():
                        out_dma(ci - 2, b).wait()

                    @plsc.parallel_loop(0, GK, step=16, unroll=8)
                    def _vec(j):
                        iv = idx_v[b, pl.ds(j, 16)]
                        val_v[b, pl.ds(j, 16)] = plsc.load_gather(
                            col_v, [iv])

                    out_dma(ci, b).start()

                    @pl.when(ci + 2 < G_NCHUNK)
                    def _():
                        idx_dma(ci + 2, b).start()

            out_dma(G_NCHUNK - 2, 0).wait()
            out_dma(G_NCHUNK - 1, 1).wait()

    return gather_k(x3, ei3)


def _sc_scatter(upd0, upd1, upd2, src1d, zeros_row):
    mesh = plsc.VectorSubcoreMesh(core_axis_name="c", subcore_axis_name="s")

    @functools.partial(
        pl.kernel,
        out_type=jax.ShapeDtypeStruct((6, 1, NP_PAD), jnp.float32),
        mesh=mesh,
        compiler_params=pltpu.CompilerParams(use_tc_tiling_on_sc=False, needs_layout_passes=False),
        scratch_types=[
            pltpu.VMEM_SHARED((NP_PAD,), jnp.float32),
            pltpu.VMEM_SHARED((NP_PAD,), jnp.float32),
            pltpu.VMEM_SHARED((NP_PAD,), jnp.float32),
            pltpu.VMEM((2, SK), jnp.int32),
            pltpu.VMEM((2, 3, SK), jnp.float32),
            pltpu.SemaphoreType.DMA,
            pltpu.SemaphoreType.DMA,
            pltpu.SemaphoreType.DMA,
            pltpu.SemaphoreType.DMA,
            pltpu.SemaphoreType.DMA,
        ],
    )
    def scatter_k(upd0_hbm, upd1_hbm, upd2_hbm, src_hbm, zrow_hbm, out_hbm,
                  sh0, sh1, sh2, idx_v, val_v, isem0, isem1, vsem0, vsem1,
                  asem):
        cid = lax.axis_index("c")
        sid = lax.axis_index("s")
        wid = sid * NC + cid
        shared = (sh0, sh1, sh2)
        upds = (upd0_hbm, upd1_hbm, upd2_hbm)
        isems = (isem0, isem1)
        vsems = (vsem0, vsem1)

        def idx_dma(ci, b):
            return pltpu.make_async_copy(
                src_hbm.at[pl.ds(wid * S_TILE + ci * SK, SK)],
                idx_v.at[b], isems[b])

        def val_dma(ci, b, c):
            return pltpu.make_async_copy(
                upds[c].at[pl.ds(wid * S_TILE + ci * SK, SK)],
                val_v.at[b, c], vsems[b])

        for c in range(3):
            pltpu.sync_copy(zrow_hbm.at[pl.ds(sid * NZ, NZ)],
                            shared[c].at[pl.ds(sid * NZ, NZ)])
        plsc.subcore_barrier()

        for b in range(2):
            idx_dma(b, b).start()
            for c in range(3):
                val_dma(b, b, c).start()

        @pl.loop(0, S_NCHUNK // 2)
        def _pair(t):
            for b in range(2):
                ci = 2 * t + b
                idx_dma(ci, b).wait()
                adds = []
                for c in range(3):
                    val_dma(ci, b, c).wait()
                    adds.append(pltpu.make_async_copy(
                        val_v.at[b, c], shared[c].at[idx_v.at[b]], asem))
                for d in adds:
                    d.start(add=True)
                for d in adds:
                    d.wait()

                @pl.when(ci + 2 < S_NCHUNK)
                def _():
                    idx_dma(ci + 2, b).start()
                    for c in range(3):
                        val_dma(ci + 2, b, c).start()

        plsc.subcore_barrier()
        for c in range(3):
            pltpu.sync_copy(shared[c].at[pl.ds(sid * NZ, NZ)],
                            out_hbm.at[cid * 3 + c, 0, pl.ds(sid * NZ, NZ)])

    return scatter_k(upd0, upd1, upd2, src1d, zeros_row)


def _edge_features(xsd_ref, ea_ref):
    xs = xsd_ref[0:3, 0]
    xd = xsd_ref[3:6, 0]
    dif = xs - xd
    norms = _psi(jnp.sum(dif * dif, axis=0, keepdims=True))
    dots = _psi(jnp.sum(xs * xd, axis=0, keepdims=True))
    m_in = jnp.concatenate([xd, xs, ea_ref[...], norms, dots], axis=0)
    return m_in, dif


def _tc_stats(xsd, ea_t, w1t):
    def body(xsd_ref, ea_ref, w1t_ref, out_ref, nd_ref):
        i = pl.program_id(0)

        @pl.when(i == 0)
        def _():
            out_ref[...] = jnp.zeros_like(out_ref)

        m_in, _ = _edge_features(xsd_ref, ea_ref)
        nd_ref[...] = m_in[10:12]
        h0 = jnp.dot(w1t_ref[...], m_in, preferred_element_type=jnp.float32)
        out_ref[:, 0:1] += jnp.sum(h0, axis=1, keepdims=True)
        out_ref[:, 1:2] += jnp.sum(h0 * h0, axis=1, keepdims=True)

    return pl.pallas_call(
        body,
        grid=(T_GRID,),
        in_specs=[
            pl.BlockSpec((6, 1, TB), lambda i: (0, 0, i)),
            pl.BlockSpec((4, TB), lambda i: (0, i)),
            pl.BlockSpec((NH, 12), lambda i: (0, 0)),
        ],
        out_specs=[
            pl.BlockSpec((NH, 128), lambda i: (0, 0)),
            pl.BlockSpec((2, TB), lambda i: (0, i)),
        ],
        out_shape=[
            jax.ShapeDtypeStruct((NH, 128), jnp.float32),
            jax.ShapeDtypeStruct((2, N_EDGES), jnp.float32),
        ],
    )(xsd, ea_t, w1t)


def _tc_mlp(xsd, ea_t, nd, w1t, scale, shift, w2ct, b2c, wx1t, bx1, wx2t):
    def body(xsd_ref, ea_ref, nd_ref, w1t_ref, scale_ref, shift_ref, w2ct_ref,
             b2c_ref, wx1t_ref, bx1_ref, wx2t_ref, o0_ref, o1_ref, o2_ref):
        xs = xsd_ref[0:3, 0]
        xd = xsd_ref[3:6, 0]
        dif = xs - xd
        m_in = jnp.concatenate([xd, xs, ea_ref[...], nd_ref[...]], axis=0)
        h0 = jnp.dot(w1t_ref[...], m_in, preferred_element_type=jnp.float32)
        h1 = jax.nn.relu(h0 * scale_ref[...] + shift_ref[...])
        z = jnp.dot(w2ct_ref[...], h1,
                    preferred_element_type=jnp.float32) + b2c_ref[...]
        h2 = jax.nn.relu(z[0:NH])
        wgt = jax.nn.sigmoid(z[NH:NH + 1])
        m = h2 * wgt
        p = jax.nn.relu(
            jnp.dot(wx1t_ref[...], m,
                    preferred_element_type=jnp.float32) + bx1_ref[...])
        px = jnp.dot(wx2t_ref[...], p, preferred_element_type=jnp.float32)
        u = jnp.clip(dif * px, -100.0, 100.0)
        o0_ref[...] = u[0]
        o1_ref[...] = u[1]
        o2_ref[...] = u[2]

    return pl.pallas_call(
        body,
        grid=(T_GRID2,),
        in_specs=[
            pl.BlockSpec((6, 1, TB2), lambda i: (0, 0, i)),
            pl.BlockSpec((4, TB2), lambda i: (0, i)),
            pl.BlockSpec((2, TB2), lambda i: (0, i)),
            pl.BlockSpec((NH, 12), lambda i: (0, 0)),
            pl.BlockSpec((NH, 1), lambda i: (0, 0)),
            pl.BlockSpec((NH, 1), lambda i: (0, 0)),
            pl.BlockSpec((NH + 1, NH), lambda i: (0, 0)),
            pl.BlockSpec((NH + 1, 1), lambda i: (0, 0)),
            pl.BlockSpec((NH, NH), lambda i: (0, 0)),
            pl.BlockSpec((NH, 1), lambda i: (0, 0)),
            pl.BlockSpec((1, NH), lambda i: (0, 0)),
        ],
        out_specs=[
            pl.BlockSpec((TB2,), lambda i: (i,)),
            pl.BlockSpec((TB2,), lambda i: (i,)),
            pl.BlockSpec((TB2,), lambda i: (i,)),
        ],
        out_shape=[
            jax.ShapeDtypeStruct((E_PAD,), jnp.float32),
            jax.ShapeDtypeStruct((E_PAD,), jnp.float32),
            jax.ShapeDtypeStruct((E_PAD,), jnp.float32),
        ],
    )(xsd, ea_t, nd, w1t, scale, shift, w2ct, b2c, wx1t, bx1, wx2t)


XB = 12544


def _tc_xupdate(x3, partials):
    def body(x_ref, p_ref, out_ref):
        out_ref[...] = x_ref[...] + C_WEIGHT * (
            p_ref[0:3] + p_ref[3:6])

    return pl.pallas_call(
        body,
        grid=(NP_PAD // XB,),
        in_specs=[
            pl.BlockSpec((3, 1, XB), lambda i: (0, 0, i)),
            pl.BlockSpec((6, 1, XB), lambda i: (0, 0, i)),
        ],
        out_specs=pl.BlockSpec((3, 1, XB), lambda i: (0, 0, i)),
        out_shape=jax.ShapeDtypeStruct((3, 1, N_NODES), jnp.float32),
    )(x3, partials)


def _tc_final(xsd, we1t, be1, we2t, be2, we3t, be3):
    def body(xsd_ref, we1t_ref, be1_ref, we2t_ref, be2_ref, we3t_ref,
             be3_ref, out_ref):
        cat = jnp.concatenate([xsd_ref[3:6, 0], xsd_ref[0:3, 0]], axis=0)
        o1 = jax.nn.relu(
            jnp.dot(we1t_ref[...], cat,
                    preferred_element_type=jnp.float32) + be1_ref[...])
        o2 = jax.nn.relu(
            jnp.dot(we2t_ref[...], o1,
                    preferred_element_type=jnp.float32) + be2_ref[...])
        o3 = jnp.dot(we3t_ref[...], o2,
                     preferred_element_type=jnp.float32) + be3_ref[...]
        out_ref[...] = jax.nn.sigmoid(o3)

    return pl.pallas_call(
        body,
        grid=(T_GRID,),
        in_specs=[
            pl.BlockSpec((6, 1, TB), lambda i: (0, 0, i)),
            pl.BlockSpec((NH, 6), lambda i: (0, 0)),
            pl.BlockSpec((NH, 1), lambda i: (0, 0)),
            pl.BlockSpec((NH, NH), lambda i: (0, 0)),
            pl.BlockSpec((NH, 1), lambda i: (0, 0)),
            pl.BlockSpec((1, NH), lambda i: (0, 0)),
            pl.BlockSpec((1, 1), lambda i: (0, 0)),
        ],
        out_specs=pl.BlockSpec((1, TB), lambda i: (0, i)),
        out_shape=jax.ShapeDtypeStruct((1, N_EDGES), jnp.float32),
    )(xsd, we1t, be1, we2t, be2, we3t, be3)


def kernel(x, edge_index, edge_attr, W1, b1, gamma, beta, W2, b2, Wm, bm,
           Wx1, bx1, Wx2, We1, be1, We2, be2, We3, be3):
    f32 = jnp.float32
    x3 = x.T[:, None, :]
    ei3 = edge_index[:, None, :]
    ea_t = edge_attr.T
    zeros_row = jnp.zeros((NP_PAD,), f32)
    src1d = jnp.concatenate(
        [edge_index[0], jnp.full((E_PAD - N_EDGES,), N_NODES, jnp.int32)])

    for l in range(N_LAYERS):
        xsd = _sc_gather(x3, ei3)
        w1t = W1[l].T
        mom, nd = _tc_stats(xsd, ea_t, w1t)
        s1 = mom[:, 0:1] / N_EDGES
        s2 = mom[:, 1:2] / N_EDGES
        mu = s1 + b1[l][:, None]
        var = s2 - s1 * s1
        scale = gamma[l][:, None] * lax.rsqrt(var + 1e-5)
        shift = beta[l][:, None] - (mu * scale)
        w2ct = jnp.concatenate([W2[l], Wm[l]], axis=1).T
        b2c = jnp.concatenate([b2[l], bm[l]])[:, None]
        upd0, upd1, upd2 = _tc_mlp(xsd, ea_t, nd, w1t, scale, shift, w2ct,
                                   b2c, Wx1[l].T, bx1[l][:, None], Wx2[l].T)
        partials = _sc_scatter(upd0, upd1, upd2, src1d, zeros_row)
        x3 = _tc_xupdate(x3, partials)

    xsd = _sc_gather(x3, ei3)
    out = _tc_final(xsd, We1.T, be1[:, None], We2.T, be2[:, None],
                    We3.T, be3[None, :])
    return out.reshape(N_EDGES, 1)
